# Initial kernel scaffold; baseline (speedup 1.0000x reference)
import jax
import jax.numpy as jnp
from jax.experimental import pallas as pl


def kernel(x, edge_index, batch, W1, b1, W2, b2, W3, b3, Wl, bl):
    raise NotImplementedError("write your pallas kernel here")


# speedup vs baseline: 19.4602x; 19.4602x over previous
import functools

import jax
import jax.numpy as jnp
from jax import lax
from jax.experimental import pallas as pl
from jax.experimental.pallas import tpu as pltpu
from jax.experimental.pallas import tpu_sc as plsc

N = 10000
F = 128
H = 128
G = 64

NC = 2
NS = 16
NW = NC * NS

NPAD = 10240
STRIPE = NPAD // NS
CHUNK = 128
E = 320000
CPW = -(-E // (NW * CHUNK))
EPAD = NW * CPW * CHUNK
BLK = 512
NBLK = NPAD // BLK

_sc_mesh = plsc.VectorSubcoreMesh(core_axis_name="c", subcore_axis_name="s")
_sc_params = pltpu.CompilerParams(needs_layout_passes=False)


def _deg_body(dst_hbm, out_hbm, dst_v, deg_v):
    c = lax.axis_index("c")
    s = lax.axis_index("s")
    wid = c * NS + s
    zero16 = jnp.zeros((16,), jnp.float32)

    def zbody(i, carry):
        for t in range(CHUNK // 16):
            deg_v[i, pl.ds(t * 16, 16)] = zero16
        return carry

    lax.fori_loop(0, NPAD // CHUNK, zbody, 0)
    pltpu.sync_copy(dst_hbm.at[wid], dst_v)
    ones16 = jnp.ones((16,), jnp.float32)

    def hbody(j, carry):
        for t in range(CHUNK // 16):
            idx = dst_v[j, pl.ds(t * 16, 16)]
            hi = lax.shift_right_logical(idx, 7)
            lo = lax.bitwise_and(idx, 127)
            plsc.addupdate_scatter(deg_v, [hi, lo], ones16)
        return carry

    lax.fori_loop(0, CPW, hbody, 0)
    pltpu.sync_copy(deg_v, out_hbm.at[wid])


_deg_call = pl.kernel(
    _deg_body,
    out_type=jax.ShapeDtypeStruct((NW, NPAD // CHUNK, CHUNK), jnp.float32),
    mesh=_sc_mesh,
    scratch_types=[
        pltpu.VMEM((CPW, CHUNK), jnp.int32),
        pltpu.VMEM((NPAD // CHUNK, CHUNK), jnp.float32),
    ],
    compiler_params=_sc_params,
)


def _agg_body(y_hbm, src_hbm, dst_hbm, out_hbm, src_v, dst_v, buf, agg_sh, sem):
    c = lax.axis_index("c")
    s = lax.axis_index("s")
    wid = c * NS + s
    zero16 = jnp.zeros((16,), jnp.float32)

    def zbody(i, carry):
        for t in range(H // 16):
            buf[i, pl.ds(t * 16, 16)] = zero16
        return carry

    lax.fori_loop(0, CHUNK, zbody, 0)
    base = s * STRIPE
    for i in range(STRIPE // CHUNK):
        pltpu.sync_copy(buf, agg_sh.at[pl.ds(base + i * CHUNK, CHUNK)])
    plsc.subcore_barrier()

    pltpu.sync_copy(src_hbm.at[wid], src_v)
    pltpu.sync_copy(dst_hbm.at[wid], dst_v)

    def body(j, carry):
        pltpu.async_copy(y_hbm.at[src_v.at[j]], buf, sem).wait()
        pltpu.sync_copy(buf, agg_sh.at[dst_v.at[j]], add=True)
        return carry

    lax.fori_loop(0, CPW, body, 0)
    plsc.subcore_barrier()
    pltpu.sync_copy(agg_sh.at[pl.ds(base, STRIPE)],
                    out_hbm.at[pl.ds(c * NPAD + base, STRIPE)])


_agg_call = pl.kernel(
    _agg_body,
    out_type=jax.ShapeDtypeStruct((NC * NPAD, H), jnp.float32),
    mesh=_sc_mesh,
    scratch_types=[
        pltpu.VMEM((CPW, CHUNK), jnp.int32),
        pltpu.VMEM((CPW, CHUNK), jnp.int32),
        pltpu.VMEM((CHUNK, H), jnp.float32),
        pltpu.VMEM_SHARED((NPAD, H), jnp.float32),
        pltpu.SemaphoreType.DMA,
    ],
    compiler_params=_sc_params,
)


def _dinv_of(deg_blk):
    deg = jnp.sum(deg_blk, axis=0) + 1.0
    return lax.rsqrt(jnp.maximum(deg, 1.0))


def _tc1_body(deg_ref, x_ref, w_ref, y_ref):
    dinv = _dinv_of(deg_ref[...])
    xw = jnp.dot(x_ref[...], w_ref[...], preferred_element_type=jnp.float32)
    y_ref[...] = xw * dinv[:, None]


def _tc_mid_body(deg_ref, a0_ref, a1_ref, y_ref, b_ref, w_ref, o_ref, *, relu):
    dinv = _dinv_of(deg_ref[...])
    h = dinv[:, None] * (a0_ref[...] + a1_ref[...] + y_ref[...]) + b_ref[...]
    if relu:
        h = jnp.maximum(h, 0.0)
    hw = jnp.dot(h, w_ref[...], preferred_element_type=jnp.float32)
    o_ref[...] = hw * dinv[:, None]


def _tc4_body(deg_ref, a0_ref, a1_ref, y_ref, b_ref, batch_ref, wl_ref, bl_ref,
              o_ref, sums_ref, cnts_ref):
    i = pl.program_id(0)

    @pl.when(i == 0)
    def _():
        sums_ref[...] = jnp.zeros_like(sums_ref)
        cnts_ref[...] = jnp.zeros_like(cnts_ref)

    dinv = _dinv_of(deg_ref[...])
    h = dinv[:, None] * (a0_ref[...] + a1_ref[...] + y_ref[...]) + b_ref[...]
    seg = batch_ref[0:1, :]
    iota = lax.broadcasted_iota(jnp.int32, (G, BLK), 0)
    onehot_t = jnp.where(iota == seg, 1.0, 0.0)
    sums_ref[...] += jax.lax.dot_general(
        onehot_t, h, (((1,), (0,)), ((), ())),
        preferred_element_type=jnp.float32)
    cnts_ref[...] += jax.lax.dot_general(
        onehot_t, jnp.ones((BLK, 1), jnp.float32), (((1,), (0,)), ((), ())),
        preferred_element_type=jnp.float32)

    @pl.when(i == NBLK - 1)
    def _():
        pooled = sums_ref[...] / jnp.maximum(cnts_ref[...], 1.0)
        o_ref[...] = jnp.dot(pooled, wl_ref[...],
                             preferred_element_type=jnp.float32) + bl_ref[...]


_deg_spec = pl.BlockSpec((NW, BLK), lambda i: (0, i))
_row_spec = pl.BlockSpec((BLK, H), lambda i: (i, 0))
_w_spec = pl.BlockSpec((H, H), lambda i: (0, 0))
_b_spec = pl.BlockSpec((1, H), lambda i: (0, 0))

_tc1_call = pl.pallas_call(
    _tc1_body,
    grid=(NBLK,),
    in_specs=[_deg_spec, _row_spec, _w_spec],
    out_specs=_row_spec,
    out_shape=jax.ShapeDtypeStruct((NPAD, H), jnp.float32),
)


def _tc_mid_call(relu):
    return pl.pallas_call(
        functools.partial(_tc_mid_body, relu=relu),
        grid=(NBLK,),
        in_specs=[_deg_spec, _row_spec, _row_spec, _row_spec, _b_spec, _w_spec],
        out_specs=_row_spec,
        out_shape=jax.ShapeDtypeStruct((NPAD, H), jnp.float32),
    )


_tc_mid_relu = _tc_mid_call(True)

_tc4_call = pl.pallas_call(
    _tc4_body,
    grid=(NBLK,),
    in_specs=[
        _deg_spec, _row_spec, _row_spec, _row_spec, _b_spec,
        pl.BlockSpec((8, BLK), lambda i: (0, i)),
        pl.BlockSpec((H, 1), lambda i: (0, 0)),
        pl.BlockSpec((1, 1), lambda i: (0, 0)),
    ],
    out_specs=pl.BlockSpec((G, 1), lambda i: (0, 0)),
    out_shape=jax.ShapeDtypeStruct((G, 1), jnp.float32),
    scratch_shapes=[
        pltpu.VMEM((G, H), jnp.float32),
        pltpu.VMEM((G, 1), jnp.float32),
    ],
)


def kernel(x, edge_index, batch, W1, b1, W2, b2, W3, b3, Wl, bl):
    x = x.astype(jnp.float32)
    src = edge_index[0]
    dst = edge_index[1]
    npad = EPAD - E
    ar = jnp.arange(npad, dtype=jnp.int32)
    pad_src = ar % N
    pad_dst = N + ar % (NPAD - N)
    src_p = jnp.concatenate([src, pad_src]).reshape(NW, CPW, CHUNK)
    dst_p = jnp.concatenate([dst, pad_dst]).reshape(NW, CPW, CHUNK)
    x_p = jnp.pad(x, ((0, NPAD - N), (0, 0)))
    batch_rep = jnp.broadcast_to(
        jnp.pad(batch, (0, NPAD - N), constant_values=G)[None, :], (8, NPAD))
    b1r = b1.reshape(1, H)
    b2r = b2.reshape(1, H)
    b3r = b3.reshape(1, H)
    blr = bl.reshape(1, 1)

    deg_part = _deg_call(dst_p).reshape(NW, NPAD)

    y1 = _tc1_call(deg_part, x_p, W1)
    agg1 = _agg_call(y1, src_p, dst_p)
    y2 = _tc_mid_relu(deg_part, agg1[:NPAD], agg1[NPAD:], y1, b1r, W2)
    agg2 = _agg_call(y2, src_p, dst_p)
    y3 = _tc_mid_relu(deg_part, agg2[:NPAD], agg2[NPAD:], y2, b2r, W3)
    agg3 = _agg_call(y3, src_p, dst_p)
    return _tc4_call(deg_part, agg3[:NPAD], agg3[NPAD:], y3, b3r, batch_rep,
                     Wl, blr)


# speedup vs baseline: 22.7147x; 1.1672x over previous
import functools

import jax
import jax.numpy as jnp
from jax import lax
from jax.experimental import pallas as pl
from jax.experimental.pallas import tpu as pltpu
from jax.experimental.pallas import tpu_sc as plsc

N = 10000
F = 128
H = 128
G = 64

NC = 2
NS = 16
NW = NC * NS

NPAD = 10240
STRIPE = NPAD // NS
CHUNK = 64
E = 320000
CPW = 160
CPW_G = CPW + 2
EPAD = NW * CPW * CHUNK
BLK = 512
NBLK = NPAD // BLK

_sc_mesh = plsc.VectorSubcoreMesh(core_axis_name="c", subcore_axis_name="s")
_sc_params = pltpu.CompilerParams(needs_layout_passes=False)


def _deg_body(pk_hbm, out_hbm, r0, r1, deg_v, sem0, sem1):
    c = lax.axis_index("c")
    s = lax.axis_index("s")
    wid = c * NS + s
    base = wid * CPW_G * CHUNK
    zero16 = jnp.zeros((16,), jnp.float32)

    def zbody(i, carry):
        for t in range(8):
            deg_v[i, pl.ds(t * 16, 16)] = zero16
        return carry

    lax.fori_loop(0, NPAD // 128, zbody, 0)
    ones16 = jnp.ones((16,), jnp.float32)
    pltpu.async_copy(pk_hbm.at[pl.ds(base, CHUNK)], r0, sem0)
    pltpu.async_copy(pk_hbm.at[pl.ds(base + CHUNK, CHUNK)], r1, sem1)

    def scatter_slot(ring):
        for t in range(CHUNK // 16):
            d = lax.bitwise_and(ring[pl.ds(t * 16, 16)], 16383)
            hi = lax.shift_right_logical(d, 7)
            lo = lax.bitwise_and(d, 127)
            plsc.addupdate_scatter(deg_v, [hi, lo], ones16)

    def hbody(k, carry):
        j0 = 2 * k
        pltpu.make_async_copy(pk_hbm.at[pl.ds(base, CHUNK)], r0, sem0).wait()
        scatter_slot(r0)
        pltpu.async_copy(pk_hbm.at[pl.ds(base + (j0 + 2) * CHUNK, CHUNK)],
                         r0, sem0)
        pltpu.make_async_copy(pk_hbm.at[pl.ds(base, CHUNK)], r1, sem1).wait()
        scatter_slot(r1)
        pltpu.async_copy(pk_hbm.at[pl.ds(base + (j0 + 3) * CHUNK, CHUNK)],
                         r1, sem1)
        return carry

    lax.fori_loop(0, CPW // 2, hbody, 0)
    pltpu.make_async_copy(pk_hbm.at[pl.ds(base, CHUNK)], r0, sem0).wait()
    pltpu.make_async_copy(pk_hbm.at[pl.ds(base, CHUNK)], r1, sem1).wait()
    pltpu.sync_copy(deg_v, out_hbm.at[wid])


_deg_call = pl.kernel(
    _deg_body,
    out_type=jax.ShapeDtypeStruct((NW, NPAD // 128, 128), jnp.float32),
    mesh=_sc_mesh,
    scratch_types=[
        pltpu.VMEM((CHUNK,), jnp.int32),
        pltpu.VMEM((CHUNK,), jnp.int32),
        pltpu.VMEM((NPAD // 128, 128), jnp.float32),
        pltpu.SemaphoreType.DMA,
        pltpu.SemaphoreType.DMA,
    ],
    compiler_params=_sc_params,
)


def _agg_body(y_hbm, pk_hbm, out_hbm, pk_v, sr0, dr0, sr1, dr1, buf0, buf1,
              agg_sh, sem0, sem1):
    c = lax.axis_index("c")
    s = lax.axis_index("s")
    wid = c * NS + s
    zero16 = jnp.zeros((16,), jnp.float32)

    def zbody(i, carry):
        for t in range(H // 16):
            buf0[i, pl.ds(t * 16, 16)] = zero16
        return carry

    lax.fori_loop(0, CHUNK, zbody, 0)
    base = s * STRIPE
    for i in range(STRIPE // CHUNK):
        pltpu.sync_copy(buf0, agg_sh.at[pl.ds(base + i * CHUNK, CHUNK)])
    plsc.subcore_barrier()

    pltpu.sync_copy(pk_hbm.at[wid], pk_v)

    def unpack(j, sring, dring):
        for t in range(CHUNK // 16):
            v = pk_v[j, pl.ds(t * 16, 16)]
            sring[pl.ds(t * 16, 16)] = lax.shift_right_logical(v, 14)
            dring[pl.ds(t * 16, 16)] = lax.bitwise_and(v, 16383)

    unpack(0, sr0, dr0)
    unpack(1, sr1, dr1)
    pltpu.async_copy(y_hbm.at[sr0], buf0, sem0)
    pltpu.async_copy(y_hbm.at[sr1], buf1, sem1)

    def body(k, carry):
        j0 = 2 * k
        pltpu.make_async_copy(y_hbm.at[sr0], buf0, sem0).wait()
        pltpu.sync_copy(buf0, agg_sh.at[dr0], add=True)
        unpack(j0 + 2, sr0, dr0)
        pltpu.async_copy(y_hbm.at[sr0], buf0, sem0)
        pltpu.make_async_copy(y_hbm.at[sr1], buf1, sem1).wait()
        pltpu.sync_copy(buf1, agg_sh.at[dr1], add=True)
        unpack(j0 + 3, sr1, dr1)
        pltpu.async_copy(y_hbm.at[sr1], buf1, sem1)
        return carry

    lax.fori_loop(0, CPW // 2, body, 0)
    pltpu.make_async_copy(y_hbm.at[sr0], buf0, sem0).wait()
    pltpu.make_async_copy(y_hbm.at[sr1], buf1, sem1).wait()
    plsc.subcore_barrier()
    pltpu.sync_copy(agg_sh.at[pl.ds(base, STRIPE)],
                    out_hbm.at[pl.ds(c * NPAD + base, STRIPE)])


_agg_call = pl.kernel(
    _agg_body,
    out_type=jax.ShapeDtypeStruct((NC * NPAD, H), jnp.float32),
    mesh=_sc_mesh,
    scratch_types=[
        pltpu.VMEM((CPW_G, CHUNK), jnp.int32),
        pltpu.VMEM((CHUNK,), jnp.int32),
        pltpu.VMEM((CHUNK,), jnp.int32),
        pltpu.VMEM((CHUNK,), jnp.int32),
        pltpu.VMEM((CHUNK,), jnp.int32),
        pltpu.VMEM((CHUNK, H), jnp.float32),
        pltpu.VMEM((CHUNK, H), jnp.float32),
        pltpu.VMEM_SHARED((NPAD, H), jnp.float32),
        pltpu.SemaphoreType.DMA,
        pltpu.SemaphoreType.DMA,
    ],
    compiler_params=_sc_params,
)


def _dinv_of(deg_blk):
    deg = jnp.sum(deg_blk, axis=0) + 1.0
    return lax.rsqrt(jnp.maximum(deg, 1.0))


def _tc1_body(deg_ref, x_ref, w_ref, y_ref):
    dinv = _dinv_of(deg_ref[...])
    xw = jnp.dot(x_ref[...], w_ref[...], preferred_element_type=jnp.float32)
    y_ref[...] = xw * dinv[:, None]


def _tc_mid_body(deg_ref, a0_ref, a1_ref, y_ref, b_ref, w_ref, o_ref, *, relu):
    dinv = _dinv_of(deg_ref[...])
    h = dinv[:, None] * (a0_ref[...] + a1_ref[...] + y_ref[...]) + b_ref[...]
    if relu:
        h = jnp.maximum(h, 0.0)
    hw = jnp.dot(h, w_ref[...], preferred_element_type=jnp.float32)
    o_ref[...] = hw * dinv[:, None]


def _tc4_body(deg_ref, a0_ref, a1_ref, y_ref, b_ref, batch_ref, wl_ref, bl_ref,
              o_ref, sums_ref, cnts_ref):
    i = pl.program_id(0)

    @pl.when(i == 0)
    def _():
        sums_ref[...] = jnp.zeros_like(sums_ref)
        cnts_ref[...] = jnp.zeros_like(cnts_ref)

    dinv = _dinv_of(deg_ref[...])
    h = dinv[:, None] * (a0_ref[...] + a1_ref[...] + y_ref[...]) + b_ref[...]
    seg = batch_ref[0:1, :]
    iota = lax.broadcasted_iota(jnp.int32, (G, BLK), 0)
    onehot_t = jnp.where(iota == seg, 1.0, 0.0)
    sums_ref[...] += jax.lax.dot_general(
        onehot_t, h, (((1,), (0,)), ((), ())),
        preferred_element_type=jnp.float32)
    cnts_ref[...] += jax.lax.dot_general(
        onehot_t, jnp.ones((BLK, 1), jnp.float32), (((1,), (0,)), ((), ())),
        preferred_element_type=jnp.float32)

    @pl.when(i == NBLK - 1)
    def _():
        pooled = sums_ref[...] / jnp.maximum(cnts_ref[...], 1.0)
        o_ref[...] = jnp.dot(pooled, wl_ref[...],
                             preferred_element_type=jnp.float32) + bl_ref[...]


_deg_spec = pl.BlockSpec((NW, BLK), lambda i: (0, i))
_row_spec = pl.BlockSpec((BLK, H), lambda i: (i, 0))
_w_spec = pl.BlockSpec((H, H), lambda i: (0, 0))
_b_spec = pl.BlockSpec((1, H), lambda i: (0, 0))

_tc1_call = pl.pallas_call(
    _tc1_body,
    grid=(NBLK,),
    in_specs=[_deg_spec, _row_spec, _w_spec],
    out_specs=_row_spec,
    out_shape=jax.ShapeDtypeStruct((NPAD, H), jnp.float32),
)


def _tc_mid_call(relu):
    return pl.pallas_call(
        functools.partial(_tc_mid_body, relu=relu),
        grid=(NBLK,),
        in_specs=[_deg_spec, _row_spec, _row_spec, _row_spec, _b_spec, _w_spec],
        out_specs=_row_spec,
        out_shape=jax.ShapeDtypeStruct((NPAD, H), jnp.float32),
    )


_tc_mid_relu = _tc_mid_call(True)

_tc4_call = pl.pallas_call(
    _tc4_body,
    grid=(NBLK,),
    in_specs=[
        _deg_spec, _row_spec, _row_spec, _row_spec, _b_spec,
        pl.BlockSpec((8, BLK), lambda i: (0, i)),
        pl.BlockSpec((H, 1), lambda i: (0, 0)),
        pl.BlockSpec((1, 1), lambda i: (0, 0)),
    ],
    out_specs=pl.BlockSpec((G, 1), lambda i: (0, 0)),
    out_shape=jax.ShapeDtypeStruct((G, 1), jnp.float32),
    scratch_shapes=[
        pltpu.VMEM((G, H), jnp.float32),
        pltpu.VMEM((G, 1), jnp.float32),
    ],
)


def kernel(x, edge_index, batch, W1, b1, W2, b2, W3, b3, Wl, bl):
    x = x.astype(jnp.float32)
    src = edge_index[0]
    dst = edge_index[1]
    npad = EPAD - E
    ar = jnp.arange(npad, dtype=jnp.int32)
    pad_src = ar % N
    pad_dst = N + ar % (NPAD - N)
    ar2 = jnp.arange(NW * 2 * CHUNK, dtype=jnp.int32)
    src_p = jnp.concatenate([
        jnp.concatenate([src, pad_src]).reshape(NW, CPW, CHUNK),
        (ar2 % N).reshape(NW, 2, CHUNK),
    ], axis=1)
    dst_p = jnp.concatenate([
        jnp.concatenate([dst, pad_dst]).reshape(NW, CPW, CHUNK),
        (N + ar2 % (NPAD - N)).reshape(NW, 2, CHUNK),
    ], axis=1)
    pk_p = src_p * 16384 + dst_p
    x_p = jnp.pad(x, ((0, NPAD - N), (0, 0)))
    batch_rep = jnp.broadcast_to(
        jnp.pad(batch, (0, NPAD - N), constant_values=G)[None, :], (8, NPAD))
    b1r = b1.reshape(1, H)
    b2r = b2.reshape(1, H)
    b3r = b3.reshape(1, H)
    blr = bl.reshape(1, 1)

    deg_part = _deg_call(pk_p.reshape(-1)).reshape(NW, NPAD)

    y1 = _tc1_call(deg_part, x_p, W1)
    agg1 = _agg_call(y1, pk_p)
    y2 = _tc_mid_relu(deg_part, agg1[:NPAD], agg1[NPAD:], y1, b1r, W2)
    agg2 = _agg_call(y2, pk_p)
    y3 = _tc_mid_relu(deg_part, agg2[:NPAD], agg2[NPAD:], y2, b2r, W3)
    agg3 = _agg_call(y3, pk_p)
    return _tc4_call(deg_part, agg3[:NPAD], agg3[NPAD:], y3, b3r, batch_rep,
                     Wl, blr)


# speedup vs baseline: 23.9499x; 1.0544x over previous
import functools

import jax
import jax.numpy as jnp
from jax import lax
from jax.experimental import pallas as pl
from jax.experimental.pallas import tpu as pltpu
from jax.experimental.pallas import tpu_sc as plsc

N = 10000
F = 128
H = 128
G = 64

NC = 2
NS = 16
NW = NC * NS

NPAD = 10240
STRIPE = NPAD // NS
CHUNK = 64
E = 320000
CPW = 160
CPW_G = CPW + 2
EPAD = NW * CPW * CHUNK
BLK = 512
NBLK = NPAD // BLK

_sc_mesh = plsc.VectorSubcoreMesh(core_axis_name="c", subcore_axis_name="s")
_sc_params = pltpu.CompilerParams(needs_layout_passes=False)


QTR = CPW // 4


def _deg_body(pk_hbm, out_hbm, qbuf, deg_v, sem):
    c = lax.axis_index("c")
    s = lax.axis_index("s")
    wid = c * NS + s
    base = wid * CPW_G * CHUNK
    zero16 = jnp.zeros((16,), jnp.float32)

    def zbody(i, carry):
        for t in range(8):
            deg_v[i, pl.ds(t * 16, 16)] = zero16
        return carry

    lax.fori_loop(0, NPAD // 128, zbody, 0)
    ones16 = jnp.ones((16,), jnp.float32)
    pltpu.async_copy(pk_hbm.at[pl.ds(base, QTR * CHUNK)], qbuf, sem)

    def hbody(j, carry):
        for t in range(CHUNK // 16):
            d = lax.bitwise_and(qbuf[pl.ds(j * CHUNK + t * 16, 16)], 16383)
            hi = lax.shift_right_logical(d, 7)
            lo = lax.bitwise_and(d, 127)
            plsc.addupdate_scatter(deg_v, [hi, lo], ones16)
        return carry

    for q in range(4):
        pltpu.make_async_copy(pk_hbm.at[pl.ds(base, QTR * CHUNK)], qbuf,
                              sem).wait()
        lax.fori_loop(0, QTR, hbody, 0)
        if q < 3:
            pltpu.async_copy(
                pk_hbm.at[pl.ds(base + (q + 1) * QTR * CHUNK, QTR * CHUNK)],
                qbuf, sem)
    pltpu.sync_copy(deg_v, out_hbm.at[wid])


_deg_call = pl.kernel(
    _deg_body,
    out_type=jax.ShapeDtypeStruct((NW, NPAD // 128, 128), jnp.float32),
    mesh=_sc_mesh,
    scratch_types=[
        pltpu.VMEM((QTR * CHUNK,), jnp.int32),
        pltpu.VMEM((NPAD // 128, 128), jnp.float32),
        pltpu.SemaphoreType.DMA,
    ],
    compiler_params=_sc_params,
)


def _agg_body(y_hbm, pk_hbm, out_hbm, pk_v, sr0, dr0, sr1, dr1, buf0, buf1,
              agg_sh, sem0, sem1):
    c = lax.axis_index("c")
    s = lax.axis_index("s")
    wid = c * NS + s
    zero16 = jnp.zeros((16,), jnp.float32)

    def zbody(i, carry):
        for t in range(H // 16):
            buf0[i, pl.ds(t * 16, 16)] = zero16
        return carry

    lax.fori_loop(0, CHUNK, zbody, 0)
    base = s * STRIPE
    for i in range(STRIPE // CHUNK):
        pltpu.sync_copy(buf0, agg_sh.at[pl.ds(base + i * CHUNK, CHUNK)])
    plsc.subcore_barrier()

    pltpu.sync_copy(pk_hbm.at[wid], pk_v)

    def unpack(j, sring, dring):
        for t in range(CHUNK // 16):
            v = pk_v[j, pl.ds(t * 16, 16)]
            sring[pl.ds(t * 16, 16)] = lax.shift_right_logical(v, 14)
            dring[pl.ds(t * 16, 16)] = lax.bitwise_and(v, 16383)

    unpack(0, sr0, dr0)
    unpack(1, sr1, dr1)
    pltpu.async_copy(y_hbm.at[sr0], buf0, sem0)
    pltpu.async_copy(y_hbm.at[sr1], buf1, sem1)

    def body(k, carry):
        j0 = 2 * k
        pltpu.make_async_copy(y_hbm.at[sr0], buf0, sem0).wait()
        pltpu.sync_copy(buf0, agg_sh.at[dr0], add=True)
        unpack(j0 + 2, sr0, dr0)
        pltpu.async_copy(y_hbm.at[sr0], buf0, sem0)
        pltpu.make_async_copy(y_hbm.at[sr1], buf1, sem1).wait()
        pltpu.sync_copy(buf1, agg_sh.at[dr1], add=True)
        unpack(j0 + 3, sr1, dr1)
        pltpu.async_copy(y_hbm.at[sr1], buf1, sem1)
        return carry

    lax.fori_loop(0, CPW // 2, body, 0)
    pltpu.make_async_copy(y_hbm.at[sr0], buf0, sem0).wait()
    pltpu.make_async_copy(y_hbm.at[sr1], buf1, sem1).wait()
    plsc.subcore_barrier()
    pltpu.sync_copy(agg_sh.at[pl.ds(base, STRIPE)],
                    out_hbm.at[pl.ds(c * NPAD + base, STRIPE)])


_agg_call = pl.kernel(
    _agg_body,
    out_type=jax.ShapeDtypeStruct((NC * NPAD, H), jnp.float32),
    mesh=_sc_mesh,
    scratch_types=[
        pltpu.VMEM((CPW_G, CHUNK), jnp.int32),
        pltpu.VMEM((CHUNK,), jnp.int32),
        pltpu.VMEM((CHUNK,), jnp.int32),
        pltpu.VMEM((CHUNK,), jnp.int32),
        pltpu.VMEM((CHUNK,), jnp.int32),
        pltpu.VMEM((CHUNK, H), jnp.float32),
        pltpu.VMEM((CHUNK, H), jnp.float32),
        pltpu.VMEM_SHARED((NPAD, H), jnp.float32),
        pltpu.SemaphoreType.DMA,
        pltpu.SemaphoreType.DMA,
    ],
    compiler_params=_sc_params,
)


def _dinv_of(deg_blk):
    deg = jnp.sum(deg_blk, axis=0) + 1.0
    return lax.rsqrt(jnp.maximum(deg, 1.0))


def _tc1_body(deg_ref, x_ref, w_ref, y_ref):
    dinv = _dinv_of(deg_ref[...])
    xw = jnp.dot(x_ref[...], w_ref[...], preferred_element_type=jnp.float32)
    y_ref[...] = xw * dinv[:, None]


def _tc_mid_body(deg_ref, a0_ref, a1_ref, y_ref, b_ref, w_ref, o_ref, *, relu):
    dinv = _dinv_of(deg_ref[...])
    h = dinv[:, None] * (a0_ref[...] + a1_ref[...] + y_ref[...]) + b_ref[...]
    if relu:
        h = jnp.maximum(h, 0.0)
    hw = jnp.dot(h, w_ref[...], preferred_element_type=jnp.float32)
    o_ref[...] = hw * dinv[:, None]


def _tc4_body(deg_ref, a0_ref, a1_ref, y_ref, b_ref, batch_ref, wl_ref, bl_ref,
              o_ref, sums_ref, cnts_ref):
    i = pl.program_id(0)

    @pl.when(i == 0)
    def _():
        sums_ref[...] = jnp.zeros_like(sums_ref)
        cnts_ref[...] = jnp.zeros_like(cnts_ref)

    dinv = _dinv_of(deg_ref[...])
    h = dinv[:, None] * (a0_ref[...] + a1_ref[...] + y_ref[...]) + b_ref[...]
    seg = batch_ref[0:1, :]
    iota = lax.broadcasted_iota(jnp.int32, (G, BLK), 0)
    onehot_t = jnp.where(iota == seg, 1.0, 0.0)
    sums_ref[...] += jax.lax.dot_general(
        onehot_t, h, (((1,), (0,)), ((), ())),
        preferred_element_type=jnp.float32)
    cnts_ref[...] += jax.lax.dot_general(
        onehot_t, jnp.ones((BLK, 1), jnp.float32), (((1,), (0,)), ((), ())),
        preferred_element_type=jnp.float32)

    @pl.when(i == NBLK - 1)
    def _():
        pooled = sums_ref[...] / jnp.maximum(cnts_ref[...], 1.0)
        o_ref[...] = jnp.dot(pooled, wl_ref[...],
                             preferred_element_type=jnp.float32) + bl_ref[...]


_deg_spec = pl.BlockSpec((NW, BLK), lambda i: (0, i))
_row_spec = pl.BlockSpec((BLK, H), lambda i: (i, 0))
_w_spec = pl.BlockSpec((H, H), lambda i: (0, 0))
_b_spec = pl.BlockSpec((1, H), lambda i: (0, 0))

_tc1_call = pl.pallas_call(
    _tc1_body,
    grid=(NBLK,),
    in_specs=[_deg_spec, _row_spec, _w_spec],
    out_specs=_row_spec,
    out_shape=jax.ShapeDtypeStruct((NPAD, H), jnp.float32),
)


def _tc_mid_call(relu):
    return pl.pallas_call(
        functools.partial(_tc_mid_body, relu=relu),
        grid=(NBLK,),
        in_specs=[_deg_spec, _row_spec, _row_spec, _row_spec, _b_spec, _w_spec],
        out_specs=_row_spec,
        out_shape=jax.ShapeDtypeStruct((NPAD, H), jnp.float32),
    )


_tc_mid_relu = _tc_mid_call(True)

_tc4_call = pl.pallas_call(
    _tc4_body,
    grid=(NBLK,),
    in_specs=[
        _deg_spec, _row_spec, _row_spec, _row_spec, _b_spec,
        pl.BlockSpec((8, BLK), lambda i: (0, i)),
        pl.BlockSpec((H, 1), lambda i: (0, 0)),
        pl.BlockSpec((1, 1), lambda i: (0, 0)),
    ],
    out_specs=pl.BlockSpec((G, 1), lambda i: (0, 0)),
    out_shape=jax.ShapeDtypeStruct((G, 1), jnp.float32),
    scratch_shapes=[
        pltpu.VMEM((G, H), jnp.float32),
        pltpu.VMEM((G, 1), jnp.float32),
    ],
)


def kernel(x, edge_index, batch, W1, b1, W2, b2, W3, b3, Wl, bl):
    x = x.astype(jnp.float32)
    src = edge_index[0]
    dst = edge_index[1]
    npad = EPAD - E
    ar = jnp.arange(npad, dtype=jnp.int32)
    pad_src = ar % N
    pad_dst = N + ar % (NPAD - N)
    ar2 = jnp.arange(NW * 2 * CHUNK, dtype=jnp.int32)
    src_p = jnp.concatenate([
        jnp.concatenate([src, pad_src]).reshape(NW, CPW, CHUNK),
        (ar2 % N).reshape(NW, 2, CHUNK),
    ], axis=1)
    dst_p = jnp.concatenate([
        jnp.concatenate([dst, pad_dst]).reshape(NW, CPW, CHUNK),
        (N + ar2 % (NPAD - N)).reshape(NW, 2, CHUNK),
    ], axis=1)
    pk_p = src_p * 16384 + dst_p
    x_p = jnp.pad(x, ((0, NPAD - N), (0, 0)))
    batch_rep = jnp.broadcast_to(
        jnp.pad(batch, (0, NPAD - N), constant_values=G)[None, :], (8, NPAD))
    b1r = b1.reshape(1, H)
    b2r = b2.reshape(1, H)
    b3r = b3.reshape(1, H)
    blr = bl.reshape(1, 1)

    deg_part = _deg_call(pk_p.reshape(-1)).reshape(NW, NPAD)

    y1 = _tc1_call(deg_part, x_p, W1)
    agg1 = _agg_call(y1, pk_p)
    y2 = _tc_mid_relu(deg_part, agg1[:NPAD], agg1[NPAD:], y1, b1r, W2)
    agg2 = _agg_call(y2, pk_p)
    y3 = _tc_mid_relu(deg_part, agg2[:NPAD], agg2[NPAD:], y2, b2r, W3)
    agg3 = _agg_call(y3, pk_p)
    return _tc4_call(deg_part, agg3[:NPAD], agg3[NPAD:], y3, b3r, batch_rep,
                     Wl, blr)


# speedup vs baseline: 27.3886x; 1.1436x over previous
import functools

import jax
import jax.numpy as jnp
from jax import lax
from jax.experimental import pallas as pl
from jax.experimental.pallas import tpu as pltpu
from jax.experimental.pallas import tpu_sc as plsc

N = 10000
F = 128
H = 128
G = 64

NC = 2
NS = 16
NW = NC * NS

NPAD = 10240
STRIPE = NPAD // NS
CHUNK = 48
E = 320000
CPW = 212
CPW_G = CPW + 4
EPAD = NW * CPW * CHUNK
BLK = 512
NBLK = NPAD // BLK

_sc_mesh = plsc.VectorSubcoreMesh(core_axis_name="c", subcore_axis_name="s")
_sc_params = pltpu.CompilerParams(needs_layout_passes=False)


QTR = CPW // 4


def _deg_body(pk_hbm, out_hbm, qbuf, deg_v, sem):
    c = lax.axis_index("c")
    s = lax.axis_index("s")
    wid = c * NS + s
    base = wid * CPW_G * CHUNK
    zero16 = jnp.zeros((16,), jnp.float32)

    def zbody(i, carry):
        for t in range(8):
            deg_v[i, pl.ds(t * 16, 16)] = zero16
        return carry

    lax.fori_loop(0, NPAD // 128, zbody, 0)
    ones16 = jnp.ones((16,), jnp.float32)
    pltpu.async_copy(pk_hbm.at[pl.ds(base, QTR * CHUNK)], qbuf, sem)

    def hbody(j, carry):
        for t in range(CHUNK // 16):
            d = lax.bitwise_and(qbuf[pl.ds(j * CHUNK + t * 16, 16)], 16383)
            hi = lax.shift_right_logical(d, 7)
            lo = lax.bitwise_and(d, 127)
            plsc.addupdate_scatter(deg_v, [hi, lo], ones16)
        return carry

    for q in range(4):
        pltpu.make_async_copy(pk_hbm.at[pl.ds(base, QTR * CHUNK)], qbuf,
                              sem).wait()
        lax.fori_loop(0, QTR, hbody, 0)
        if q < 3:
            pltpu.async_copy(
                pk_hbm.at[pl.ds(base + (q + 1) * QTR * CHUNK, QTR * CHUNK)],
                qbuf, sem)
    pltpu.sync_copy(deg_v, out_hbm.at[wid])


_deg_call = pl.kernel(
    _deg_body,
    out_type=jax.ShapeDtypeStruct((NW, NPAD // 128, 128), jnp.float32),
    mesh=_sc_mesh,
    scratch_types=[
        pltpu.VMEM((QTR * CHUNK,), jnp.int32),
        pltpu.VMEM((NPAD // 128, 128), jnp.float32),
        pltpu.SemaphoreType.DMA,
    ],
    compiler_params=_sc_params,
)


def _agg_body(y_hbm, pk_hbm, out_hbm, pk_v,
              sr0, dr0, sr1, dr1, sr2, dr2, sr3, dr3,
              b0, b1, b2, b3, agg_sh,
              g0, g1, g2, g3, s0, s1, s2, s3):
    c = lax.axis_index("c")
    s = lax.axis_index("s")
    wid = c * NS + s
    srs = [sr0, sr1, sr2, sr3]
    drs = [dr0, dr1, dr2, dr3]
    bufs = [b0, b1, b2, b3]
    gsem = [g0, g1, g2, g3]
    ssem = [s0, s1, s2, s3]
    zero16 = jnp.zeros((16,), jnp.float32)

    def zbody(i, carry):
        for t in range(H // 16):
            b0[i, pl.ds(t * 16, 16)] = zero16
        return carry

    lax.fori_loop(0, CHUNK, zbody, 0)
    base = s * STRIPE
    off = 0
    while off < STRIPE:
        sz = min(CHUNK, STRIPE - off)
        pltpu.sync_copy(b0.at[pl.ds(0, sz)], agg_sh.at[pl.ds(base + off, sz)])
        off += sz
    plsc.subcore_barrier()

    base_pk = wid * CPW_G * CHUNK
    half = (CPW_G // 2) * CHUNK
    pltpu.sync_copy(pk_hbm.at[pl.ds(base_pk, half)], pk_v)

    def unpack(j, p, off):
        for t in range(CHUNK // 16):
            v = pk_v[pl.ds((j - off) * CHUNK + t * 16, 16)]
            srs[p][pl.ds(t * 16, 16)] = lax.shift_right_logical(v, 14)
            drs[p][pl.ds(t * 16, 16)] = lax.bitwise_and(v, 16383)

    for j in range(4):
        unpack(j, j, 0)
        pltpu.async_copy(y_hbm.at[srs[j]], bufs[j], gsem[j])
        if j >= 2:
            q = j - 2
            pltpu.make_async_copy(y_hbm.at[srs[q]], bufs[q], gsem[q]).wait()
            pltpu.async_copy(bufs[q], agg_sh.at[drs[q]], ssem[q], add=True)

    def mk_body(off):
        def body(k, carry):
            for p in range(4):
                j = 4 * k + p
                q = (p + 2) % 4
                pltpu.make_async_copy(bufs[p], agg_sh.at[drs[p]],
                                      ssem[p]).wait()
                unpack(j, p, off)
                pltpu.async_copy(y_hbm.at[srs[p]], bufs[p], gsem[p])
                pltpu.make_async_copy(y_hbm.at[srs[q]], bufs[q],
                                      gsem[q]).wait()
                pltpu.async_copy(bufs[q], agg_sh.at[drs[q]], ssem[q], add=True)
            return carry
        return body

    half_k = (CPW_G // 2) // 4
    lax.fori_loop(1, half_k, mk_body(0), 0)
    pltpu.sync_copy(pk_hbm.at[pl.ds(base_pk + half, half)], pk_v)
    lax.fori_loop(half_k, CPW // 4 + 1, mk_body(CPW_G // 2), 0)
    pltpu.make_async_copy(bufs[0], agg_sh.at[drs[0]], ssem[0]).wait()
    pltpu.make_async_copy(bufs[1], agg_sh.at[drs[1]], ssem[1]).wait()
    pltpu.make_async_copy(y_hbm.at[srs[2]], bufs[2], gsem[2]).wait()
    pltpu.make_async_copy(y_hbm.at[srs[3]], bufs[3], gsem[3]).wait()
    plsc.subcore_barrier()
    pltpu.sync_copy(agg_sh.at[pl.ds(base, STRIPE)],
                    out_hbm.at[pl.ds(c * NPAD + base, STRIPE)])


_agg_call = pl.kernel(
    _agg_body,
    out_type=jax.ShapeDtypeStruct((NC * NPAD, H), jnp.float32),
    mesh=_sc_mesh,
    scratch_types=(
        [pltpu.VMEM(((CPW_G // 2) * CHUNK,), jnp.int32)]
        + [pltpu.VMEM((CHUNK,), jnp.int32) for _ in range(8)]
        + [pltpu.VMEM((CHUNK, H), jnp.float32) for _ in range(4)]
        + [pltpu.VMEM_SHARED((NPAD, H), jnp.float32)]
        + [pltpu.SemaphoreType.DMA for _ in range(8)]
    ),
    compiler_params=_sc_params,
)


def _dinv_of(deg_blk):
    deg = jnp.sum(deg_blk, axis=0) + 1.0
    return lax.rsqrt(jnp.maximum(deg, 1.0))


def _tc1_body(deg_ref, x_ref, w_ref, y_ref):
    dinv = _dinv_of(deg_ref[...])
    xw = jnp.dot(x_ref[...], w_ref[...], preferred_element_type=jnp.float32)
    y_ref[...] = xw * dinv[:, None]


def _tc_mid_body(deg_ref, a0_ref, a1_ref, y_ref, b_ref, w_ref, o_ref, *, relu):
    dinv = _dinv_of(deg_ref[...])
    h = dinv[:, None] * (a0_ref[...] + a1_ref[...] + y_ref[...]) + b_ref[...]
    if relu:
        h = jnp.maximum(h, 0.0)
    hw = jnp.dot(h, w_ref[...], preferred_element_type=jnp.float32)
    o_ref[...] = hw * dinv[:, None]


def _tc4_body(deg_ref, a0_ref, a1_ref, y_ref, b_ref, batch_ref, wl_ref, bl_ref,
              o_ref, sums_ref, cnts_ref):
    i = pl.program_id(0)

    @pl.when(i == 0)
    def _():
        sums_ref[...] = jnp.zeros_like(sums_ref)
        cnts_ref[...] = jnp.zeros_like(cnts_ref)

    dinv = _dinv_of(deg_ref[...])
    h = dinv[:, None] * (a0_ref[...] + a1_ref[...] + y_ref[...]) + b_ref[...]
    seg = batch_ref[0:1, :]
    iota = lax.broadcasted_iota(jnp.int32, (G, BLK), 0)
    onehot_t = jnp.where(iota == seg, 1.0, 0.0)
    sums_ref[...] += jax.lax.dot_general(
        onehot_t, h, (((1,), (0,)), ((), ())),
        preferred_element_type=jnp.float32)
    cnts_ref[...] += jax.lax.dot_general(
        onehot_t, jnp.ones((BLK, 1), jnp.float32), (((1,), (0,)), ((), ())),
        preferred_element_type=jnp.float32)

    @pl.when(i == NBLK - 1)
    def _():
        pooled = sums_ref[...] / jnp.maximum(cnts_ref[...], 1.0)
        o_ref[...] = jnp.dot(pooled, wl_ref[...],
                             preferred_element_type=jnp.float32) + bl_ref[...]


_deg_spec = pl.BlockSpec((NW, BLK), lambda i: (0, i))
_row_spec = pl.BlockSpec((BLK, H), lambda i: (i, 0))
_w_spec = pl.BlockSpec((H, H), lambda i: (0, 0))
_b_spec = pl.BlockSpec((1, H), lambda i: (0, 0))

_tc1_call = pl.pallas_call(
    _tc1_body,
    grid=(NBLK,),
    in_specs=[_deg_spec, _row_spec, _w_spec],
    out_specs=_row_spec,
    out_shape=jax.ShapeDtypeStruct((NPAD, H), jnp.float32),
)


def _tc_mid_call(relu):
    return pl.pallas_call(
        functools.partial(_tc_mid_body, relu=relu),
        grid=(NBLK,),
        in_specs=[_deg_spec, _row_spec, _row_spec, _row_spec, _b_spec, _w_spec],
        out_specs=_row_spec,
        out_shape=jax.ShapeDtypeStruct((NPAD, H), jnp.float32),
    )


_tc_mid_relu = _tc_mid_call(True)

_tc4_call = pl.pallas_call(
    _tc4_body,
    grid=(NBLK,),
    in_specs=[
        _deg_spec, _row_spec, _row_spec, _row_spec, _b_spec,
        pl.BlockSpec((8, BLK), lambda i: (0, i)),
        pl.BlockSpec((H, 1), lambda i: (0, 0)),
        pl.BlockSpec((1, 1), lambda i: (0, 0)),
    ],
    out_specs=pl.BlockSpec((G, 1), lambda i: (0, 0)),
    out_shape=jax.ShapeDtypeStruct((G, 1), jnp.float32),
    scratch_shapes=[
        pltpu.VMEM((G, H), jnp.float32),
        pltpu.VMEM((G, 1), jnp.float32),
    ],
)


def kernel(x, edge_index, batch, W1, b1, W2, b2, W3, b3, Wl, bl):
    x = x.astype(jnp.float32)
    src = edge_index[0]
    dst = edge_index[1]
    npad = EPAD - E
    ar = jnp.arange(npad, dtype=jnp.int32)
    pad_src = ar % N
    pad_dst = N + ar % (NPAD - N)
    ar2 = jnp.arange(NW * 4 * CHUNK, dtype=jnp.int32)
    src_p = jnp.concatenate([
        jnp.concatenate([src, pad_src]).reshape(NW, CPW, CHUNK),
        (ar2 % N).reshape(NW, 4, CHUNK),
    ], axis=1)
    dst_p = jnp.concatenate([
        jnp.concatenate([dst, pad_dst]).reshape(NW, CPW, CHUNK),
        (N + ar2 % (NPAD - N)).reshape(NW, 4, CHUNK),
    ], axis=1)
    pk_p = src_p * 16384 + dst_p
    x_p = jnp.pad(x, ((0, NPAD - N), (0, 0)))
    batch_rep = jnp.broadcast_to(
        jnp.pad(batch, (0, NPAD - N), constant_values=G)[None, :], (8, NPAD))
    b1r = b1.reshape(1, H)
    b2r = b2.reshape(1, H)
    b3r = b3.reshape(1, H)
    blr = bl.reshape(1, 1)

    deg_part = _deg_call(pk_p.reshape(-1)).reshape(NW, NPAD)

    pk_f = pk_p.reshape(-1)
    y1 = _tc1_call(deg_part, x_p, W1)
    agg1 = _agg_call(y1, pk_f)
    y2 = _tc_mid_relu(deg_part, agg1[:NPAD], agg1[NPAD:], y1, b1r, W2)
    agg2 = _agg_call(y2, pk_f)
    y3 = _tc_mid_relu(deg_part, agg2[:NPAD], agg2[NPAD:], y2, b2r, W3)
    agg3 = _agg_call(y3, pk_f)
    return _tc4_call(deg_part, agg3[:NPAD], agg3[NPAD:], y3, b3r, batch_rep,
                     Wl, blr)


# speedup vs baseline: 28.7937x; 1.0513x over previous
import functools

import jax
import jax.numpy as jnp
from jax import lax
from jax.experimental import pallas as pl
from jax.experimental.pallas import tpu as pltpu
from jax.experimental.pallas import tpu_sc as plsc

N = 10000
F = 128
H = 128
G = 64

NC = 2
NS = 16
NW = NC * NS

NPAD = 10240
STRIPE = NPAD // NS
CHUNK = 48
E = 320000
CPW = 212
CPW_G = CPW + 4
EPAD = NW * CPW * CHUNK
BLK = 512
NBLK = NPAD // BLK

_sc_mesh = plsc.VectorSubcoreMesh(core_axis_name="c", subcore_axis_name="s")
_sc_params = pltpu.CompilerParams(needs_layout_passes=False)


QTR = CPW // 4


def _deg_body(pk_hbm, out_hbm, qbuf, deg_v, sem):
    c = lax.axis_index("c")
    s = lax.axis_index("s")
    wid = c * NS + s
    base = wid * CPW_G * CHUNK
    zero16 = jnp.zeros((16,), jnp.float32)

    def zbody(i, carry):
        for t in range(8):
            deg_v[i, pl.ds(t * 16, 16)] = zero16
        return carry

    lax.fori_loop(0, NPAD // 128, zbody, 0)
    ones16 = jnp.ones((16,), jnp.float32)
    pltpu.async_copy(pk_hbm.at[pl.ds(base, QTR * CHUNK)], qbuf, sem)

    def hbody(j, carry):
        for t in range(CHUNK // 16):
            d = lax.bitwise_and(qbuf[pl.ds(j * CHUNK + t * 16, 16)], 16383)
            hi = lax.shift_right_logical(d, 7)
            lo = lax.bitwise_and(d, 127)
            plsc.addupdate_scatter(deg_v, [hi, lo], ones16)
        return carry

    for q in range(4):
        pltpu.make_async_copy(pk_hbm.at[pl.ds(base, QTR * CHUNK)], qbuf,
                              sem).wait()
        lax.fori_loop(0, QTR, hbody, 0)
        if q < 3:
            pltpu.async_copy(
                pk_hbm.at[pl.ds(base + (q + 1) * QTR * CHUNK, QTR * CHUNK)],
                qbuf, sem)
    pltpu.sync_copy(deg_v, out_hbm.at[wid])


_deg_call = pl.kernel(
    _deg_body,
    out_type=jax.ShapeDtypeStruct((NW, NPAD // 128, 128), jnp.float32),
    mesh=_sc_mesh,
    scratch_types=[
        pltpu.VMEM((QTR * CHUNK,), jnp.int32),
        pltpu.VMEM((NPAD // 128, 128), jnp.float32),
        pltpu.SemaphoreType.DMA,
    ],
    compiler_params=_sc_params,
)


def _agg_body(y_hbm, pk_hbm, out_hbm, pk_v,
              sr0, dr0, sr1, dr1, sr2, dr2, sr3, dr3,
              b0, b1, b2, b3, agg_sh,
              g0, g1, g2, g3, s0, s1, s2, s3):
    c = lax.axis_index("c")
    s = lax.axis_index("s")
    wid = c * NS + s
    srs = [sr0, sr1, sr2, sr3]
    drs = [dr0, dr1, dr2, dr3]
    bufs = [b0, b1, b2, b3]
    gsem = [g0, g1, g2, g3]
    ssem = [s0, s1, s2, s3]
    zero16 = jnp.zeros((16,), jnp.float32)

    def zbody(i, carry):
        for t in range(H // 16):
            b0[i, pl.ds(t * 16, 16)] = zero16
        return carry

    lax.fori_loop(0, CHUNK, zbody, 0)
    base = s * STRIPE
    off = 0
    while off < STRIPE:
        sz = min(CHUNK, STRIPE - off)
        pltpu.sync_copy(b0.at[pl.ds(0, sz)], agg_sh.at[pl.ds(base + off, sz)])
        off += sz
    plsc.subcore_barrier()

    base_pk = wid * CPW_G * CHUNK
    half = (CPW_G // 2) * CHUNK
    pltpu.sync_copy(pk_hbm.at[pl.ds(base_pk, half)], pk_v)

    def unpack(j, p, off):
        for t in range(CHUNK // 16):
            v = pk_v[pl.ds((j - off) * CHUNK + t * 16, 16)]
            srs[p][pl.ds(t * 16, 16)] = lax.shift_right_logical(v, 14)
            drs[p][pl.ds(t * 16, 16)] = lax.bitwise_and(v, 16383)

    for j in range(4):
        unpack(j, j, 0)
        pltpu.async_copy(y_hbm.at[srs[j]], bufs[j], gsem[j])
        if j >= 2:
            q = j - 2
            pltpu.make_async_copy(y_hbm.at[srs[q]], bufs[q], gsem[q]).wait()
            pltpu.async_copy(bufs[q], agg_sh.at[drs[q]], ssem[q], add=True)

    def mk_body(off):
        def body(k, carry):
            for p in range(4):
                j = 4 * k + p
                q = (p + 2) % 4
                pltpu.make_async_copy(bufs[p], agg_sh.at[drs[p]],
                                      ssem[p]).wait()
                unpack(j, p, off)
                pltpu.async_copy(y_hbm.at[srs[p]], bufs[p], gsem[p])
                pltpu.make_async_copy(y_hbm.at[srs[q]], bufs[q],
                                      gsem[q]).wait()
                pltpu.async_copy(bufs[q], agg_sh.at[drs[q]], ssem[q], add=True)
            return carry
        return body

    half_k = (CPW_G // 2) // 4
    lax.fori_loop(1, half_k, mk_body(0), 0)
    pltpu.sync_copy(pk_hbm.at[pl.ds(base_pk + half, half)], pk_v)
    lax.fori_loop(half_k, CPW // 4 + 1, mk_body(CPW_G // 2), 0)
    pltpu.make_async_copy(bufs[0], agg_sh.at[drs[0]], ssem[0]).wait()
    pltpu.make_async_copy(bufs[1], agg_sh.at[drs[1]], ssem[1]).wait()
    pltpu.make_async_copy(y_hbm.at[srs[2]], bufs[2], gsem[2]).wait()
    pltpu.make_async_copy(y_hbm.at[srs[3]], bufs[3], gsem[3]).wait()
    plsc.subcore_barrier()
    pltpu.sync_copy(agg_sh.at[pl.ds(base, STRIPE)],
                    out_hbm.at[pl.ds(c * NPAD + base, STRIPE)])


_agg_call = pl.kernel(
    _agg_body,
    out_type=jax.ShapeDtypeStruct((NC * NPAD, H), jnp.float32),
    mesh=_sc_mesh,
    scratch_types=(
        [pltpu.VMEM(((CPW_G // 2) * CHUNK,), jnp.int32)]
        + [pltpu.VMEM((CHUNK,), jnp.int32) for _ in range(8)]
        + [pltpu.VMEM((CHUNK, H), jnp.float32) for _ in range(4)]
        + [pltpu.VMEM_SHARED((NPAD, H), jnp.float32)]
        + [pltpu.SemaphoreType.DMA for _ in range(8)]
    ),
    compiler_params=_sc_params,
)


def _tc1_body(deg_ref, x_ref, w_ref, y_ref, dinv_ref):
    deg = jnp.sum(deg_ref[...], axis=0) + 1.0
    dinv = lax.rsqrt(jnp.maximum(deg, 1.0))
    xw = jnp.dot(x_ref[...], w_ref[...], preferred_element_type=jnp.float32)
    y_ref[...] = xw * dinv[:, None]
    dinv_ref[...] = jnp.broadcast_to(dinv[None, :], (8, BLK))


def _dinv_of(dinv_blk):
    return jnp.sum(dinv_blk, axis=0) * 0.125


def _tc_mid_body(dinv_ref, a0_ref, a1_ref, y_ref, b_ref, w_ref, o_ref, *,
                 relu):
    dinv = _dinv_of(dinv_ref[...])
    h = dinv[:, None] * (a0_ref[...] + a1_ref[...] + y_ref[...]) + b_ref[...]
    if relu:
        h = jnp.maximum(h, 0.0)
    hw = jnp.dot(h, w_ref[...], preferred_element_type=jnp.float32)
    o_ref[...] = hw * dinv[:, None]


def _tc4_body(dinv_ref, a0_ref, a1_ref, y_ref, b_ref, batch_ref, wl_ref,
              bl_ref, o_ref, sums_ref, cnts_ref):
    i = pl.program_id(0)

    @pl.when(i == 0)
    def _():
        sums_ref[...] = jnp.zeros_like(sums_ref)
        cnts_ref[...] = jnp.zeros_like(cnts_ref)

    dinv = _dinv_of(dinv_ref[...])
    h = dinv[:, None] * (a0_ref[...] + a1_ref[...] + y_ref[...]) + b_ref[...]
    seg = batch_ref[0:1, :]
    iota = lax.broadcasted_iota(jnp.int32, (G, BLK), 0)
    onehot_t = jnp.where(iota == seg, 1.0, 0.0)
    sums_ref[...] += jax.lax.dot_general(
        onehot_t, h, (((1,), (0,)), ((), ())),
        preferred_element_type=jnp.float32)
    cnts_ref[...] += jax.lax.dot_general(
        onehot_t, jnp.ones((BLK, 1), jnp.float32), (((1,), (0,)), ((), ())),
        preferred_element_type=jnp.float32)

    @pl.when(i == NBLK - 1)
    def _():
        pooled = sums_ref[...] / jnp.maximum(cnts_ref[...], 1.0)
        o_ref[...] = jnp.dot(pooled, wl_ref[...],
                             preferred_element_type=jnp.float32) + bl_ref[...]


_deg_spec = pl.BlockSpec((NW, BLK), lambda i: (0, i))
_dinv_spec = pl.BlockSpec((8, BLK), lambda i: (0, i))
_row_spec = pl.BlockSpec((BLK, H), lambda i: (i, 0))
_a0_spec = pl.BlockSpec((BLK, H), lambda i: (i, 0))
_a1_spec = pl.BlockSpec((BLK, H), lambda i: (i + NBLK, 0))
_w_spec = pl.BlockSpec((H, H), lambda i: (0, 0))
_b_spec = pl.BlockSpec((1, H), lambda i: (0, 0))

_tc1_call = pl.pallas_call(
    _tc1_body,
    grid=(NBLK,),
    in_specs=[_deg_spec, _row_spec, _w_spec],
    out_specs=[_row_spec, _dinv_spec],
    out_shape=[jax.ShapeDtypeStruct((NPAD, H), jnp.float32),
               jax.ShapeDtypeStruct((8, NPAD), jnp.float32)],
)


def _tc_mid_call(relu):
    return pl.pallas_call(
        functools.partial(_tc_mid_body, relu=relu),
        grid=(NBLK,),
        in_specs=[_dinv_spec, _a0_spec, _a1_spec, _row_spec, _b_spec, _w_spec],
        out_specs=_row_spec,
        out_shape=jax.ShapeDtypeStruct((NPAD, H), jnp.float32),
    )


_tc_mid_relu = _tc_mid_call(True)

_tc4_call = pl.pallas_call(
    _tc4_body,
    grid=(NBLK,),
    in_specs=[
        _dinv_spec, _a0_spec, _a1_spec, _row_spec, _b_spec,
        pl.BlockSpec((8, BLK), lambda i: (0, i)),
        pl.BlockSpec((H, 1), lambda i: (0, 0)),
        pl.BlockSpec((1, 1), lambda i: (0, 0)),
    ],
    out_specs=pl.BlockSpec((G, 1), lambda i: (0, 0)),
    out_shape=jax.ShapeDtypeStruct((G, 1), jnp.float32),
    scratch_shapes=[
        pltpu.VMEM((G, H), jnp.float32),
        pltpu.VMEM((G, 1), jnp.float32),
    ],
)


def kernel(x, edge_index, batch, W1, b1, W2, b2, W3, b3, Wl, bl):
    x = x.astype(jnp.float32)
    src = edge_index[0]
    dst = edge_index[1]
    npad = EPAD - E
    ar = jnp.arange(npad, dtype=jnp.int32)
    pad_src = ar % N
    pad_dst = N + ar % (NPAD - N)
    ar2 = jnp.arange(NW * 4 * CHUNK, dtype=jnp.int32)
    src_p = jnp.concatenate([
        jnp.concatenate([src, pad_src]).reshape(NW, CPW, CHUNK),
        (ar2 % N).reshape(NW, 4, CHUNK),
    ], axis=1)
    dst_p = jnp.concatenate([
        jnp.concatenate([dst, pad_dst]).reshape(NW, CPW, CHUNK),
        (N + ar2 % (NPAD - N)).reshape(NW, 4, CHUNK),
    ], axis=1)
    pk_p = src_p * 16384 + dst_p
    x_p = jnp.pad(x, ((0, NPAD - N), (0, 0)))
    batch_rep = jnp.broadcast_to(
        jnp.pad(batch, (0, NPAD - N), constant_values=G)[None, :], (8, NPAD))
    b1r = b1.reshape(1, H)
    b2r = b2.reshape(1, H)
    b3r = b3.reshape(1, H)
    blr = bl.reshape(1, 1)

    deg_part = _deg_call(pk_p.reshape(-1)).reshape(NW, NPAD)

    pk_f = pk_p.reshape(-1)
    y1, dinv_rep = _tc1_call(deg_part, x_p, W1)
    agg1 = _agg_call(y1, pk_f)
    y2 = _tc_mid_relu(dinv_rep, agg1, agg1, y1, b1r, W2)
    agg2 = _agg_call(y2, pk_f)
    y3 = _tc_mid_relu(dinv_rep, agg2, agg2, y2, b2r, W3)
    agg3 = _agg_call(y3, pk_f)
    return _tc4_call(dinv_rep, agg3, agg3, y3, b3r, batch_rep, Wl, blr)


# speedup vs baseline: 29.7240x; 1.0323x over previous
import functools

import jax
import jax.numpy as jnp
from jax import lax
from jax.experimental import pallas as pl
from jax.experimental.pallas import tpu as pltpu
from jax.experimental.pallas import tpu_sc as plsc

N = 10000
F = 128
H = 128
G = 64

NC = 2
NS = 16
NW = NC * NS

NPAD = 10240
STRIPE = NPAD // NS
CHUNK = 48
E = 320000
CPW = 212
CPW_G = CPW + 4
EPAD = NW * CPW * CHUNK
BLK = 512
NBLK = NPAD // BLK

_sc_mesh = plsc.VectorSubcoreMesh(core_axis_name="c", subcore_axis_name="s")
_sc_params = pltpu.CompilerParams(needs_layout_passes=False)


QTR = CPW // 4


def _deg_body(pk_hbm, out_hbm, qbuf, deg_v, sem):
    c = lax.axis_index("c")
    s = lax.axis_index("s")
    wid = c * NS + s
    base = wid * CPW * CHUNK
    zero16 = jnp.zeros((16,), jnp.float32)

    def zbody(i, carry):
        for t in range(8):
            deg_v[i, pl.ds(t * 16, 16)] = zero16
        return carry

    lax.fori_loop(0, NPAD // 128, zbody, 0)
    ones16 = jnp.ones((16,), jnp.float32)
    pltpu.async_copy(pk_hbm.at[pl.ds(base, QTR * CHUNK)], qbuf, sem)

    def hbody(j, carry):
        for t in range(CHUNK // 16):
            d = lax.bitwise_and(qbuf[pl.ds(j * CHUNK + t * 16, 16)], 16383)
            hi = lax.shift_right_logical(d, 7)
            lo = lax.bitwise_and(d, 127)
            plsc.addupdate_scatter(deg_v, [hi, lo], ones16)
        return carry

    for q in range(4):
        pltpu.make_async_copy(pk_hbm.at[pl.ds(base, QTR * CHUNK)], qbuf,
                              sem).wait()
        lax.fori_loop(0, QTR, hbody, 0)
        if q < 3:
            pltpu.async_copy(
                pk_hbm.at[pl.ds(base + (q + 1) * QTR * CHUNK, QTR * CHUNK)],
                qbuf, sem)
    pltpu.sync_copy(deg_v, out_hbm.at[wid])


_deg_call = pl.kernel(
    _deg_body,
    out_type=jax.ShapeDtypeStruct((NW, NPAD // 128, 128), jnp.float32),
    mesh=_sc_mesh,
    scratch_types=[
        pltpu.VMEM((QTR * CHUNK,), jnp.int32),
        pltpu.VMEM((NPAD // 128, 128), jnp.float32),
        pltpu.SemaphoreType.DMA,
    ],
    compiler_params=_sc_params,
)


def _agg_body(y_hbm, pk_hbm, out_hbm, pk_v,
              sr0, dr0, sr1, dr1, sr2, dr2, sr3, dr3,
              b0, b1, b2, b3, agg_sh,
              g0, g1, g2, g3, s0, s1, s2, s3):
    c = lax.axis_index("c")
    s = lax.axis_index("s")
    wid = c * NS + s
    srs = [sr0, sr1, sr2, sr3]
    drs = [dr0, dr1, dr2, dr3]
    bufs = [b0, b1, b2, b3]
    gsem = [g0, g1, g2, g3]
    ssem = [s0, s1, s2, s3]
    zero16 = jnp.zeros((16,), jnp.float32)

    def zbody(i, carry):
        for t in range(H // 16):
            b0[i, pl.ds(t * 16, 16)] = zero16
        return carry

    lax.fori_loop(0, CHUNK, zbody, 0)
    base = s * STRIPE
    off = 0
    while off < STRIPE:
        sz = min(CHUNK, STRIPE - off)
        pltpu.sync_copy(b0.at[pl.ds(0, sz)], agg_sh.at[pl.ds(base + off, sz)])
        off += sz
    plsc.subcore_barrier()

    base_pk = wid * CPW * CHUNK
    half = (CPW_G // 2) * CHUNK
    real2 = (CPW - CPW_G // 2) * CHUNK
    pltpu.sync_copy(pk_hbm.at[pl.ds(base_pk, half)], pk_v)

    def unpack(j, p, off):
        for t in range(CHUNK // 16):
            v = pk_v[pl.ds((j - off) * CHUNK + t * 16, 16)]
            srs[p][pl.ds(t * 16, 16)] = lax.shift_right_logical(v, 14)
            drs[p][pl.ds(t * 16, 16)] = lax.bitwise_and(v, 16383)

    for j in range(4):
        unpack(j, j, 0)
        pltpu.async_copy(y_hbm.at[srs[j]], bufs[j], gsem[j])
        if j >= 2:
            q = j - 2
            pltpu.make_async_copy(y_hbm.at[srs[q]], bufs[q], gsem[q]).wait()
            pltpu.async_copy(bufs[q], agg_sh.at[drs[q]], ssem[q], add=True)

    def mk_body(off):
        def body(k, carry):
            for p in range(4):
                j = 4 * k + p
                q = (p + 2) % 4
                pltpu.make_async_copy(bufs[p], agg_sh.at[drs[p]],
                                      ssem[p]).wait()
                unpack(j, p, off)
                pltpu.async_copy(y_hbm.at[srs[p]], bufs[p], gsem[p])
                pltpu.make_async_copy(y_hbm.at[srs[q]], bufs[q],
                                      gsem[q]).wait()
                pltpu.async_copy(bufs[q], agg_sh.at[drs[q]], ssem[q], add=True)
            return carry
        return body

    half_k = (CPW_G // 2) // 4
    lax.fori_loop(1, half_k, mk_body(0), 0)
    pltpu.sync_copy(pk_hbm.at[pl.ds(base_pk + half, real2)],
                    pk_v.at[pl.ds(0, real2)])
    pltpu.sync_copy(pk_hbm.at[pl.ds(EPAD + wid * 4 * CHUNK, 4 * CHUNK)],
                    pk_v.at[pl.ds(real2, 4 * CHUNK)])
    lax.fori_loop(half_k, CPW // 4 + 1, mk_body(CPW_G // 2), 0)
    pltpu.make_async_copy(bufs[0], agg_sh.at[drs[0]], ssem[0]).wait()
    pltpu.make_async_copy(bufs[1], agg_sh.at[drs[1]], ssem[1]).wait()
    pltpu.make_async_copy(y_hbm.at[srs[2]], bufs[2], gsem[2]).wait()
    pltpu.make_async_copy(y_hbm.at[srs[3]], bufs[3], gsem[3]).wait()
    plsc.subcore_barrier()
    pltpu.sync_copy(agg_sh.at[pl.ds(base, STRIPE)],
                    out_hbm.at[pl.ds(c * NPAD + base, STRIPE)])


_agg_call = pl.kernel(
    _agg_body,
    out_type=jax.ShapeDtypeStruct((NC * NPAD, H), jnp.float32),
    mesh=_sc_mesh,
    scratch_types=(
        [pltpu.VMEM(((CPW_G // 2) * CHUNK,), jnp.int32)]
        + [pltpu.VMEM((CHUNK,), jnp.int32) for _ in range(8)]
        + [pltpu.VMEM((CHUNK, H), jnp.float32) for _ in range(4)]
        + [pltpu.VMEM_SHARED((NPAD, H), jnp.float32)]
        + [pltpu.SemaphoreType.DMA for _ in range(8)]
    ),
    compiler_params=_sc_params,
)


def _tc1_body(deg_ref, x_ref, w_ref, y_ref, dinv_ref):
    deg = jnp.sum(deg_ref[...], axis=0) + 1.0
    dinv = lax.rsqrt(jnp.maximum(deg, 1.0))
    xw = jnp.dot(x_ref[...], w_ref[...], preferred_element_type=jnp.float32)
    y_ref[...] = xw * dinv[:, None]
    dinv_ref[...] = jnp.broadcast_to(dinv[None, :], (8, BLK))


def _dinv_of(dinv_blk):
    return jnp.sum(dinv_blk, axis=0) * 0.125


def _tc_mid_body(dinv_ref, a0_ref, a1_ref, y_ref, b_ref, w_ref, o_ref, *,
                 relu):
    dinv = _dinv_of(dinv_ref[...])
    h = dinv[:, None] * (a0_ref[...] + a1_ref[...] + y_ref[...]) + b_ref[...]
    if relu:
        h = jnp.maximum(h, 0.0)
    hw = jnp.dot(h, w_ref[...], preferred_element_type=jnp.float32)
    o_ref[...] = hw * dinv[:, None]


def _tc4_body(dinv_ref, a0_ref, a1_ref, y_ref, b_ref, batch_ref, wl_ref,
              bl_ref, o_ref, sums_ref, cnts_ref):
    i = pl.program_id(0)

    @pl.when(i == 0)
    def _():
        sums_ref[...] = jnp.zeros_like(sums_ref)
        cnts_ref[...] = jnp.zeros_like(cnts_ref)

    dinv = _dinv_of(dinv_ref[...])
    h = dinv[:, None] * (a0_ref[...] + a1_ref[...] + y_ref[...]) + b_ref[...]
    seg = batch_ref[0:1, :]
    iota = lax.broadcasted_iota(jnp.int32, (G, BLK), 0)
    onehot_t = jnp.where(iota == seg, 1.0, 0.0)
    sums_ref[...] += jax.lax.dot_general(
        onehot_t, h, (((1,), (0,)), ((), ())),
        precision=jax.lax.Precision.HIGHEST,
        preferred_element_type=jnp.float32)
    cnts_ref[...] += jax.lax.dot_general(
        onehot_t, jnp.ones((BLK, 1), jnp.float32), (((1,), (0,)), ((), ())),
        preferred_element_type=jnp.float32)

    @pl.when(i == NBLK - 1)
    def _():
        pooled = sums_ref[...] / jnp.maximum(cnts_ref[...], 1.0)
        o_ref[...] = jnp.dot(pooled, wl_ref[...],
                             preferred_element_type=jnp.float32) + bl_ref[...]


_deg_spec = pl.BlockSpec((NW, BLK), lambda i: (0, i))
_dinv_spec = pl.BlockSpec((8, BLK), lambda i: (0, i))
_row_spec = pl.BlockSpec((BLK, H), lambda i: (i, 0))
_a0_spec = pl.BlockSpec((BLK, H), lambda i: (i, 0))
_a1_spec = pl.BlockSpec((BLK, H), lambda i: (i + NBLK, 0))
_w_spec = pl.BlockSpec((H, H), lambda i: (0, 0))
_b_spec = pl.BlockSpec((1, H), lambda i: (0, 0))

_tc1_call = pl.pallas_call(
    _tc1_body,
    grid=(NBLK,),
    in_specs=[_deg_spec, _row_spec, _w_spec],
    out_specs=[_row_spec, _dinv_spec],
    out_shape=[jax.ShapeDtypeStruct((NPAD, H), jnp.float32),
               jax.ShapeDtypeStruct((8, NPAD), jnp.float32)],
)


def _tc_mid_call(relu):
    return pl.pallas_call(
        functools.partial(_tc_mid_body, relu=relu),
        grid=(NBLK,),
        in_specs=[_dinv_spec, _a0_spec, _a1_spec, _row_spec, _b_spec, _w_spec],
        out_specs=_row_spec,
        out_shape=jax.ShapeDtypeStruct((NPAD, H), jnp.float32),
    )


_tc_mid_relu = _tc_mid_call(True)

_tc4_call = pl.pallas_call(
    _tc4_body,
    grid=(NBLK,),
    in_specs=[
        _dinv_spec, _a0_spec, _a1_spec, _row_spec, _b_spec,
        pl.BlockSpec((8, BLK), lambda i: (0, i)),
        pl.BlockSpec((H, 1), lambda i: (0, 0)),
        pl.BlockSpec((1, 1), lambda i: (0, 0)),
    ],
    out_specs=pl.BlockSpec((G, 1), lambda i: (0, 0)),
    out_shape=jax.ShapeDtypeStruct((G, 1), jnp.float32),
    scratch_shapes=[
        pltpu.VMEM((G, H), jnp.float32),
        pltpu.VMEM((G, 1), jnp.float32),
    ],
)


def kernel(x, edge_index, batch, W1, b1, W2, b2, W3, b3, Wl, bl):
    x = x.astype(jnp.float32)
    npad = EPAD - E
    pk_real = jnp.sum(
        edge_index * jnp.array([16384, 1], dtype=jnp.int32)[:, None], axis=0)
    ar = jnp.arange(npad, dtype=jnp.int32)
    pk_pad = (ar % N) * 16384 + (N + ar % (NPAD - N))
    ar2 = jnp.arange(NW * 4 * CHUNK, dtype=jnp.int32)
    pk_dummy = (ar2 % N) * 16384 + (N + ar2 % (NPAD - N))
    pk_f = jnp.concatenate([pk_real, pk_pad, pk_dummy])
    x_p = jnp.pad(x, ((0, NPAD - N), (0, 0)))
    batch_rep = jnp.broadcast_to(
        jnp.pad(batch, (0, NPAD - N), constant_values=G)[None, :], (8, NPAD))
    b1r = b1.reshape(1, H)
    b2r = b2.reshape(1, H)
    b3r = b3.reshape(1, H)
    blr = bl.reshape(1, 1)

    deg_part = _deg_call(pk_f).reshape(NW, NPAD)

    y1, dinv_rep = _tc1_call(deg_part, x_p, W1)
    agg1 = _agg_call(y1, pk_f)
    y2 = _tc_mid_relu(dinv_rep, agg1, agg1, y1, b1r, W2)
    agg2 = _agg_call(y2, pk_f)
    y3 = _tc_mid_relu(dinv_rep, agg2, agg2, y2, b2r, W3)
    agg3 = _agg_call(y3, pk_f)
    return _tc4_call(dinv_rep, agg3, agg3, y3, b3r, batch_rep, Wl, blr)


# speedup vs baseline: 31.2735x; 1.0521x over previous
import functools

import jax
import jax.numpy as jnp
from jax import lax
from jax.experimental import pallas as pl
from jax.experimental.pallas import tpu as pltpu
from jax.experimental.pallas import tpu_sc as plsc

N = 10000
F = 128
H = 128
G = 64

NC = 2
NS = 16
NW = NC * NS

NPAD = 10240
STRIPE = NPAD // NS
CHUNK = 48
E = 320000
CPW = 212
CPW_G = CPW + 4
EPAD = NW * CPW * CHUNK
BLK = 1024
NBLK = NPAD // BLK

_sc_mesh = plsc.VectorSubcoreMesh(core_axis_name="c", subcore_axis_name="s")
_sc_params = pltpu.CompilerParams(needs_layout_passes=False)


QTR = CPW // 4


def _deg_body(pk_hbm, out_hbm, qbuf, deg_v, sem):
    c = lax.axis_index("c")
    s = lax.axis_index("s")
    wid = c * NS + s
    base = wid * CPW * CHUNK
    zero16 = jnp.zeros((16,), jnp.float32)

    def zbody(i, carry):
        for t in range(8):
            deg_v[i, pl.ds(t * 16, 16)] = zero16
        return carry

    lax.fori_loop(0, NPAD // 128, zbody, 0)
    ones16 = jnp.ones((16,), jnp.float32)
    pltpu.async_copy(pk_hbm.at[pl.ds(base, QTR * CHUNK)], qbuf, sem)

    def hbody(j, carry):
        for t in range(CHUNK // 16):
            d = lax.bitwise_and(qbuf[pl.ds(j * CHUNK + t * 16, 16)], 16383)
            hi = lax.shift_right_logical(d, 7)
            lo = lax.bitwise_and(d, 127)
            plsc.addupdate_scatter(deg_v, [hi, lo], ones16)
        return carry

    for q in range(4):
        pltpu.make_async_copy(pk_hbm.at[pl.ds(base, QTR * CHUNK)], qbuf,
                              sem).wait()
        lax.fori_loop(0, QTR, hbody, 0)
        if q < 3:
            pltpu.async_copy(
                pk_hbm.at[pl.ds(base + (q + 1) * QTR * CHUNK, QTR * CHUNK)],
                qbuf, sem)
    pltpu.sync_copy(deg_v, out_hbm.at[wid])


_deg_call = pl.kernel(
    _deg_body,
    out_type=jax.ShapeDtypeStruct((NW, NPAD // 128, 128), jnp.float32),
    mesh=_sc_mesh,
    scratch_types=[
        pltpu.VMEM((QTR * CHUNK,), jnp.int32),
        pltpu.VMEM((NPAD // 128, 128), jnp.float32),
        pltpu.SemaphoreType.DMA,
    ],
    compiler_params=_sc_params,
)


def _agg_body(y_hbm, pk_hbm, out_hbm, pk_v,
              sr0, dr0, sr1, dr1, sr2, dr2, sr3, dr3,
              b0, b1, b2, b3, agg_sh,
              g0, g1, g2, g3, s0, s1, s2, s3):
    c = lax.axis_index("c")
    s = lax.axis_index("s")
    wid = c * NS + s
    srs = [sr0, sr1, sr2, sr3]
    drs = [dr0, dr1, dr2, dr3]
    bufs = [b0, b1, b2, b3]
    gsem = [g0, g1, g2, g3]
    ssem = [s0, s1, s2, s3]
    zero16 = jnp.zeros((16,), jnp.float32)

    def zbody(i, carry):
        for t in range(H // 16):
            b0[i, pl.ds(t * 16, 16)] = zero16
        return carry

    lax.fori_loop(0, CHUNK, zbody, 0)
    base = s * STRIPE
    off = 0
    while off < STRIPE:
        sz = min(CHUNK, STRIPE - off)
        pltpu.sync_copy(b0.at[pl.ds(0, sz)], agg_sh.at[pl.ds(base + off, sz)])
        off += sz
    plsc.subcore_barrier()

    base_pk = wid * CPW * CHUNK
    half = (CPW_G // 2) * CHUNK
    real2 = (CPW - CPW_G // 2) * CHUNK
    pltpu.sync_copy(pk_hbm.at[pl.ds(base_pk, half)], pk_v)

    def unpack(j, p, off):
        for t in range(CHUNK // 16):
            v = pk_v[pl.ds((j - off) * CHUNK + t * 16, 16)]
            srs[p][pl.ds(t * 16, 16)] = lax.shift_right_logical(v, 14)
            drs[p][pl.ds(t * 16, 16)] = lax.bitwise_and(v, 16383)

    for j in range(4):
        unpack(j, j, 0)
        pltpu.async_copy(y_hbm.at[srs[j]], bufs[j], gsem[j])
        if j >= 2:
            q = j - 2
            pltpu.make_async_copy(y_hbm.at[srs[q]], bufs[q], gsem[q]).wait()
            pltpu.async_copy(bufs[q], agg_sh.at[drs[q]], ssem[q], add=True)

    def mk_body(off):
        def body(k, carry):
            for p in range(4):
                j = 4 * k + p
                q = (p + 2) % 4
                pltpu.make_async_copy(bufs[p], agg_sh.at[drs[p]],
                                      ssem[p]).wait()
                unpack(j, p, off)
                pltpu.async_copy(y_hbm.at[srs[p]], bufs[p], gsem[p])
                pltpu.make_async_copy(y_hbm.at[srs[q]], bufs[q],
                                      gsem[q]).wait()
                pltpu.async_copy(bufs[q], agg_sh.at[drs[q]], ssem[q], add=True)
            return carry
        return body

    half_k = (CPW_G // 2) // 4
    lax.fori_loop(1, half_k, mk_body(0), 0)
    pltpu.sync_copy(pk_hbm.at[pl.ds(base_pk + half, real2)],
                    pk_v.at[pl.ds(0, real2)])
    pltpu.sync_copy(pk_hbm.at[pl.ds(EPAD + wid * 4 * CHUNK, 4 * CHUNK)],
                    pk_v.at[pl.ds(real2, 4 * CHUNK)])
    lax.fori_loop(half_k, CPW // 4 + 1, mk_body(CPW_G // 2), 0)
    pltpu.make_async_copy(bufs[0], agg_sh.at[drs[0]], ssem[0]).wait()
    pltpu.make_async_copy(bufs[1], agg_sh.at[drs[1]], ssem[1]).wait()
    pltpu.make_async_copy(y_hbm.at[srs[2]], bufs[2], gsem[2]).wait()
    pltpu.make_async_copy(y_hbm.at[srs[3]], bufs[3], gsem[3]).wait()
    plsc.subcore_barrier()
    pltpu.sync_copy(agg_sh.at[pl.ds(base, STRIPE)],
                    out_hbm.at[pl.ds(c * NPAD + base, STRIPE)])


_agg_call = pl.kernel(
    _agg_body,
    out_type=jax.ShapeDtypeStruct((NC * NPAD, H), jnp.float32),
    mesh=_sc_mesh,
    scratch_types=(
        [pltpu.VMEM(((CPW_G // 2) * CHUNK,), jnp.int32)]
        + [pltpu.VMEM((CHUNK,), jnp.int32) for _ in range(8)]
        + [pltpu.VMEM((CHUNK, H), jnp.float32) for _ in range(4)]
        + [pltpu.VMEM_SHARED((NPAD, H), jnp.float32)]
        + [pltpu.SemaphoreType.DMA for _ in range(8)]
    ),
    compiler_params=_sc_params,
)


def _tc1_body(deg_ref, x_ref, w_ref, y_ref, dinv_ref):
    deg = jnp.sum(deg_ref[...], axis=0) + 1.0
    dinv = lax.rsqrt(jnp.maximum(deg, 1.0))
    xw = jnp.dot(x_ref[...], w_ref[...], preferred_element_type=jnp.float32)
    y_ref[...] = xw * dinv[:, None]
    dinv_ref[...] = jnp.broadcast_to(dinv[None, :], (8, BLK))


def _dinv_of(dinv_blk):
    return jnp.sum(dinv_blk, axis=0) * 0.125


def _tc_mid_body(dinv_ref, a0_ref, a1_ref, y_ref, b_ref, w_ref, o_ref, *,
                 relu):
    dinv = _dinv_of(dinv_ref[...])
    h = dinv[:, None] * (a0_ref[...] + a1_ref[...] + y_ref[...]) + b_ref[...]
    if relu:
        h = jnp.maximum(h, 0.0)
    hw = jnp.dot(h, w_ref[...], preferred_element_type=jnp.float32)
    o_ref[...] = hw * dinv[:, None]


def _tc4_body(dinv_ref, a0_ref, a1_ref, y_ref, b_ref, batch_ref, wl_ref,
              bl_ref, o_ref, sums_ref, cnts_ref):
    i = pl.program_id(0)

    @pl.when(i == 0)
    def _():
        sums_ref[...] = jnp.zeros_like(sums_ref)
        cnts_ref[...] = jnp.zeros_like(cnts_ref)

    dinv = _dinv_of(dinv_ref[...])
    h = dinv[:, None] * (a0_ref[...] + a1_ref[...] + y_ref[...]) + b_ref[...]
    seg = batch_ref[0:1, :]
    iota = lax.broadcasted_iota(jnp.int32, (G, BLK), 0)
    onehot_t = jnp.where(iota == seg, 1.0, 0.0)
    sums_ref[...] += jax.lax.dot_general(
        onehot_t, h, (((1,), (0,)), ((), ())),
        precision=jax.lax.Precision.HIGHEST,
        preferred_element_type=jnp.float32)
    cnts_ref[...] += jax.lax.dot_general(
        onehot_t, jnp.ones((BLK, 1), jnp.float32), (((1,), (0,)), ((), ())),
        preferred_element_type=jnp.float32)

    @pl.when(i == NBLK - 1)
    def _():
        pooled = sums_ref[...] / jnp.maximum(cnts_ref[...], 1.0)
        o_ref[...] = jnp.dot(pooled, wl_ref[...],
                             preferred_element_type=jnp.float32) + bl_ref[...]


_deg_spec = pl.BlockSpec((NW, BLK), lambda i: (0, i))
_dinv_spec = pl.BlockSpec((8, BLK), lambda i: (0, i))
_row_spec = pl.BlockSpec((BLK, H), lambda i: (i, 0))
_a0_spec = pl.BlockSpec((BLK, H), lambda i: (i, 0))
_a1_spec = pl.BlockSpec((BLK, H), lambda i: (i + NBLK, 0))
_w_spec = pl.BlockSpec((H, H), lambda i: (0, 0))
_b_spec = pl.BlockSpec((1, H), lambda i: (0, 0))

_tc1_call = pl.pallas_call(
    _tc1_body,
    grid=(NBLK,),
    in_specs=[_deg_spec, _row_spec, _w_spec],
    out_specs=[_row_spec, _dinv_spec],
    out_shape=[jax.ShapeDtypeStruct((NPAD, H), jnp.float32),
               jax.ShapeDtypeStruct((8, NPAD), jnp.float32)],
)


def _tc_mid_call(relu):
    return pl.pallas_call(
        functools.partial(_tc_mid_body, relu=relu),
        grid=(NBLK,),
        in_specs=[_dinv_spec, _a0_spec, _a1_spec, _row_spec, _b_spec, _w_spec],
        out_specs=_row_spec,
        out_shape=jax.ShapeDtypeStruct((NPAD, H), jnp.float32),
    )


_tc_mid_relu = _tc_mid_call(True)

_tc4_call = pl.pallas_call(
    _tc4_body,
    grid=(NBLK,),
    in_specs=[
        _dinv_spec, _a0_spec, _a1_spec, _row_spec, _b_spec,
        pl.BlockSpec((8, BLK), lambda i: (0, i)),
        pl.BlockSpec((H, 1), lambda i: (0, 0)),
        pl.BlockSpec((1, 1), lambda i: (0, 0)),
    ],
    out_specs=pl.BlockSpec((G, 1), lambda i: (0, 0)),
    out_shape=jax.ShapeDtypeStruct((G, 1), jnp.float32),
    scratch_shapes=[
        pltpu.VMEM((G, H), jnp.float32),
        pltpu.VMEM((G, 1), jnp.float32),
    ],
)


def kernel(x, edge_index, batch, W1, b1, W2, b2, W3, b3, Wl, bl):
    x = x.astype(jnp.float32)
    npad = EPAD - E
    pk_real = jnp.sum(
        edge_index * jnp.array([16384, 1], dtype=jnp.int32)[:, None], axis=0)
    ar = jnp.arange(npad, dtype=jnp.int32)
    pk_pad = (ar % N) * 16384 + (N + ar % (NPAD - N))
    ar2 = jnp.arange(NW * 4 * CHUNK, dtype=jnp.int32)
    pk_dummy = (ar2 % N) * 16384 + (N + ar2 % (NPAD - N))
    pk_f = jnp.concatenate([pk_real, pk_pad, pk_dummy])
    x_p = jnp.pad(x, ((0, NPAD - N), (0, 0)))
    batch_rep = jnp.broadcast_to(
        jnp.pad(batch, (0, NPAD - N), constant_values=G)[None, :], (8, NPAD))
    b1r = b1.reshape(1, H)
    b2r = b2.reshape(1, H)
    b3r = b3.reshape(1, H)
    blr = bl.reshape(1, 1)

    deg_part = _deg_call(pk_f).reshape(NW, NPAD)

    y1, dinv_rep = _tc1_call(deg_part, x_p, W1)
    agg1 = _agg_call(y1, pk_f)
    y2 = _tc_mid_relu(dinv_rep, agg1, agg1, y1, b1r, W2)
    agg2 = _agg_call(y2, pk_f)
    y3 = _tc_mid_relu(dinv_rep, agg2, agg2, y2, b2r, W3)
    agg3 = _agg_call(y3, pk_f)
    return _tc4_call(dinv_rep, agg3, agg3, y3, b3r, batch_rep, Wl, blr)


# speedup vs baseline: 31.9963x; 1.0231x over previous
import functools

import jax
import jax.numpy as jnp
from jax import lax
from jax.experimental import pallas as pl
from jax.experimental.pallas import tpu as pltpu
from jax.experimental.pallas import tpu_sc as plsc

N = 10000
F = 128
H = 128
G = 64

NC = 2
NS = 16
NW = NC * NS

NPAD = 10240
STRIPE = NPAD // NS
CHUNK = 48
E = 320000
CPW = 212
CPW_G = CPW + 4
EPAD = NW * CPW * CHUNK
BLK = 2048
NBLK = NPAD // BLK

_sc_mesh = plsc.VectorSubcoreMesh(core_axis_name="c", subcore_axis_name="s")
_sc_params = pltpu.CompilerParams(needs_layout_passes=False)


QTR = CPW // 4


def _deg_body(pk_hbm, out_hbm, qbuf, deg_v, sem):
    c = lax.axis_index("c")
    s = lax.axis_index("s")
    wid = c * NS + s
    base = wid * CPW * CHUNK
    zero16 = jnp.zeros((16,), jnp.float32)

    def zbody(i, carry):
        for t in range(8):
            deg_v[i, pl.ds(t * 16, 16)] = zero16
        return carry

    lax.fori_loop(0, NPAD // 128, zbody, 0)
    ones16 = jnp.ones((16,), jnp.float32)
    pltpu.async_copy(pk_hbm.at[pl.ds(base, QTR * CHUNK)], qbuf, sem)

    def hbody(j, carry):
        for t in range(CHUNK // 16):
            d = lax.bitwise_and(qbuf[pl.ds(j * CHUNK + t * 16, 16)], 16383)
            hi = lax.shift_right_logical(d, 7)
            lo = lax.bitwise_and(d, 127)
            plsc.addupdate_scatter(deg_v, [hi, lo], ones16)
        return carry

    for q in range(4):
        pltpu.make_async_copy(pk_hbm.at[pl.ds(base, QTR * CHUNK)], qbuf,
                              sem).wait()
        lax.fori_loop(0, QTR, hbody, 0)
        if q < 3:
            pltpu.async_copy(
                pk_hbm.at[pl.ds(base + (q + 1) * QTR * CHUNK, QTR * CHUNK)],
                qbuf, sem)
    pltpu.sync_copy(deg_v, out_hbm.at[wid])


_deg_call = pl.kernel(
    _deg_body,
    out_type=jax.ShapeDtypeStruct((NW, NPAD // 128, 128), jnp.float32),
    mesh=_sc_mesh,
    scratch_types=[
        pltpu.VMEM((QTR * CHUNK,), jnp.int32),
        pltpu.VMEM((NPAD // 128, 128), jnp.float32),
        pltpu.SemaphoreType.DMA,
    ],
    compiler_params=_sc_params,
)


def _agg_body(y_hbm, pk_hbm, out_hbm, pk_v,
              sr0, dr0, sr1, dr1, sr2, dr2, sr3, dr3,
              b0, b1, b2, b3, agg_sh,
              g0, g1, g2, g3, s0, s1, s2, s3):
    c = lax.axis_index("c")
    s = lax.axis_index("s")
    wid = c * NS + s
    srs = [sr0, sr1, sr2, sr3]
    drs = [dr0, dr1, dr2, dr3]
    bufs = [b0, b1, b2, b3]
    gsem = [g0, g1, g2, g3]
    ssem = [s0, s1, s2, s3]
    zero16 = jnp.zeros((16,), jnp.float32)

    def zbody(i, carry):
        for t in range(H // 16):
            b0[i, pl.ds(t * 16, 16)] = zero16
        return carry

    lax.fori_loop(0, CHUNK, zbody, 0)
    base = s * STRIPE
    off = 0
    while off < STRIPE:
        sz = min(CHUNK, STRIPE - off)
        pltpu.sync_copy(b0.at[pl.ds(0, sz)], agg_sh.at[pl.ds(base + off, sz)])
        off += sz
    plsc.subcore_barrier()

    base_pk = wid * CPW * CHUNK
    half = (CPW_G // 2) * CHUNK
    real2 = (CPW - CPW_G // 2) * CHUNK
    pltpu.sync_copy(pk_hbm.at[pl.ds(base_pk, half)], pk_v)

    def unpack(j, p, off):
        for t in range(CHUNK // 16):
            v = pk_v[pl.ds((j - off) * CHUNK + t * 16, 16)]
            srs[p][pl.ds(t * 16, 16)] = lax.shift_right_logical(v, 14)
            drs[p][pl.ds(t * 16, 16)] = lax.bitwise_and(v, 16383)

    for j in range(4):
        unpack(j, j, 0)
        pltpu.async_copy(y_hbm.at[srs[j]], bufs[j], gsem[j])
        if j >= 2:
            q = j - 2
            pltpu.make_async_copy(y_hbm.at[srs[q]], bufs[q], gsem[q]).wait()
            pltpu.async_copy(bufs[q], agg_sh.at[drs[q]], ssem[q], add=True)

    def mk_body(off):
        def body(k, carry):
            for p in range(4):
                j = 4 * k + p
                q = (p + 2) % 4
                pltpu.make_async_copy(bufs[p], agg_sh.at[drs[p]],
                                      ssem[p]).wait()
                unpack(j, p, off)
                pltpu.async_copy(y_hbm.at[srs[p]], bufs[p], gsem[p])
                pltpu.make_async_copy(y_hbm.at[srs[q]], bufs[q],
                                      gsem[q]).wait()
                pltpu.async_copy(bufs[q], agg_sh.at[drs[q]], ssem[q], add=True)
            return carry
        return body

    half_k = (CPW_G // 2) // 4
    lax.fori_loop(1, half_k, mk_body(0), 0)
    pltpu.sync_copy(pk_hbm.at[pl.ds(base_pk + half, real2)],
                    pk_v.at[pl.ds(0, real2)])
    pltpu.sync_copy(pk_hbm.at[pl.ds(EPAD + wid * 4 * CHUNK, 4 * CHUNK)],
                    pk_v.at[pl.ds(real2, 4 * CHUNK)])
    lax.fori_loop(half_k, CPW // 4 + 1, mk_body(CPW_G // 2), 0)
    pltpu.make_async_copy(bufs[0], agg_sh.at[drs[0]], ssem[0]).wait()
    pltpu.make_async_copy(bufs[1], agg_sh.at[drs[1]], ssem[1]).wait()
    pltpu.make_async_copy(y_hbm.at[srs[2]], bufs[2], gsem[2]).wait()
    pltpu.make_async_copy(y_hbm.at[srs[3]], bufs[3], gsem[3]).wait()
    plsc.subcore_barrier()
    pltpu.sync_copy(agg_sh.at[pl.ds(base, STRIPE)],
                    out_hbm.at[pl.ds(c * NPAD + base, STRIPE)])


_agg_call = pl.kernel(
    _agg_body,
    out_type=jax.ShapeDtypeStruct((NC * NPAD, H), jnp.float32),
    mesh=_sc_mesh,
    scratch_types=(
        [pltpu.VMEM(((CPW_G // 2) * CHUNK,), jnp.int32)]
        + [pltpu.VMEM((CHUNK,), jnp.int32) for _ in range(8)]
        + [pltpu.VMEM((CHUNK, H), jnp.float32) for _ in range(4)]
        + [pltpu.VMEM_SHARED((NPAD, H), jnp.float32)]
        + [pltpu.SemaphoreType.DMA for _ in range(8)]
    ),
    compiler_params=_sc_params,
)


def _tc1_body(deg_ref, x_ref, w_ref, y_ref, dinv_ref):
    deg = jnp.sum(deg_ref[...], axis=0) + 1.0
    dinv = lax.rsqrt(jnp.maximum(deg, 1.0))
    xw = jnp.dot(x_ref[...], w_ref[...], preferred_element_type=jnp.float32)
    y_ref[...] = xw * dinv[:, None]
    dinv_ref[...] = jnp.broadcast_to(dinv[None, :], (8, BLK))


def _dinv_of(dinv_blk):
    return jnp.sum(dinv_blk, axis=0) * 0.125


def _tc_mid_body(dinv_ref, a0_ref, a1_ref, y_ref, b_ref, w_ref, o_ref, *,
                 relu):
    dinv = _dinv_of(dinv_ref[...])
    h = dinv[:, None] * (a0_ref[...] + a1_ref[...] + y_ref[...]) + b_ref[...]
    if relu:
        h = jnp.maximum(h, 0.0)
    hw = jnp.dot(h, w_ref[...], preferred_element_type=jnp.float32)
    o_ref[...] = hw * dinv[:, None]


def _tc4_body(dinv_ref, a0_ref, a1_ref, y_ref, b_ref, batch_ref, wl_ref,
              bl_ref, o_ref, sums_ref, cnts_ref):
    i = pl.program_id(0)

    @pl.when(i == 0)
    def _():
        sums_ref[...] = jnp.zeros_like(sums_ref)
        cnts_ref[...] = jnp.zeros_like(cnts_ref)

    dinv = _dinv_of(dinv_ref[...])
    h = dinv[:, None] * (a0_ref[...] + a1_ref[...] + y_ref[...]) + b_ref[...]
    seg = batch_ref[0:1, :]
    iota = lax.broadcasted_iota(jnp.int32, (G, BLK), 0)
    onehot_t = jnp.where(iota == seg, 1.0, 0.0)
    sums_ref[...] += jax.lax.dot_general(
        onehot_t, h, (((1,), (0,)), ((), ())),
        precision=jax.lax.Precision.HIGHEST,
        preferred_element_type=jnp.float32)
    cnts_ref[...] += jax.lax.dot_general(
        onehot_t, jnp.ones((BLK, 1), jnp.float32), (((1,), (0,)), ((), ())),
        preferred_element_type=jnp.float32)

    @pl.when(i == NBLK - 1)
    def _():
        pooled = sums_ref[...] / jnp.maximum(cnts_ref[...], 1.0)
        o_ref[...] = jnp.dot(pooled, wl_ref[...],
                             preferred_element_type=jnp.float32) + bl_ref[...]


_deg_spec = pl.BlockSpec((NW, BLK), lambda i: (0, i))
_dinv_spec = pl.BlockSpec((8, BLK), lambda i: (0, i))
_row_spec = pl.BlockSpec((BLK, H), lambda i: (i, 0))
_a0_spec = pl.BlockSpec((BLK, H), lambda i: (i, 0))
_a1_spec = pl.BlockSpec((BLK, H), lambda i: (i + NBLK, 0))
_w_spec = pl.BlockSpec((H, H), lambda i: (0, 0))
_b_spec = pl.BlockSpec((1, H), lambda i: (0, 0))

_tc1_call = pl.pallas_call(
    _tc1_body,
    grid=(NBLK,),
    in_specs=[_deg_spec, _row_spec, _w_spec],
    out_specs=[_row_spec, _dinv_spec],
    out_shape=[jax.ShapeDtypeStruct((NPAD, H), jnp.float32),
               jax.ShapeDtypeStruct((8, NPAD), jnp.float32)],
)


def _tc_mid_call(relu):
    return pl.pallas_call(
        functools.partial(_tc_mid_body, relu=relu),
        grid=(NBLK,),
        in_specs=[_dinv_spec, _a0_spec, _a1_spec, _row_spec, _b_spec, _w_spec],
        out_specs=_row_spec,
        out_shape=jax.ShapeDtypeStruct((NPAD, H), jnp.float32),
    )


_tc_mid_relu = _tc_mid_call(True)

_tc4_call = pl.pallas_call(
    _tc4_body,
    grid=(NBLK,),
    in_specs=[
        _dinv_spec, _a0_spec, _a1_spec, _row_spec, _b_spec,
        pl.BlockSpec((8, BLK), lambda i: (0, i)),
        pl.BlockSpec((H, 1), lambda i: (0, 0)),
        pl.BlockSpec((1, 1), lambda i: (0, 0)),
    ],
    out_specs=pl.BlockSpec((G, 1), lambda i: (0, 0)),
    out_shape=jax.ShapeDtypeStruct((G, 1), jnp.float32),
    scratch_shapes=[
        pltpu.VMEM((G, H), jnp.float32),
        pltpu.VMEM((G, 1), jnp.float32),
    ],
)


def kernel(x, edge_index, batch, W1, b1, W2, b2, W3, b3, Wl, bl):
    x = x.astype(jnp.float32)
    npad = EPAD - E
    pk_real = jnp.sum(
        edge_index * jnp.array([16384, 1], dtype=jnp.int32)[:, None], axis=0)
    ar = jnp.arange(npad, dtype=jnp.int32)
    pk_pad = (ar % N) * 16384 + (N + ar % (NPAD - N))
    ar2 = jnp.arange(NW * 4 * CHUNK, dtype=jnp.int32)
    pk_dummy = (ar2 % N) * 16384 + (N + ar2 % (NPAD - N))
    pk_f = jnp.concatenate([pk_real, pk_pad, pk_dummy])
    x_p = jnp.pad(x, ((0, NPAD - N), (0, 0)))
    batch_rep = jnp.broadcast_to(
        jnp.pad(batch, (0, NPAD - N), constant_values=G)[None, :], (8, NPAD))
    b1r = b1.reshape(1, H)
    b2r = b2.reshape(1, H)
    b3r = b3.reshape(1, H)
    blr = bl.reshape(1, 1)

    deg_part = _deg_call(pk_f).reshape(NW, NPAD)

    y1, dinv_rep = _tc1_call(deg_part, x_p, W1)
    agg1 = _agg_call(y1, pk_f)
    y2 = _tc_mid_relu(dinv_rep, agg1, agg1, y1, b1r, W2)
    agg2 = _agg_call(y2, pk_f)
    y3 = _tc_mid_relu(dinv_rep, agg2, agg2, y2, b2r, W3)
    agg3 = _agg_call(y3, pk_f)
    return _tc4_call(dinv_rep, agg3, agg3, y3, b3r, batch_rep, Wl, blr)
